# baseline (speedup 1.0000x reference)
import functools

import jax
import jax.numpy as jnp
from jax import lax
from jax.experimental import pallas as pl
from jax.experimental.pallas import tpu as pltpu
from jax.experimental.pallas import tpu_sc as plsc

N = 10000
E = 320000
NC, NS, L = 2, 16, 16
NW = NC * NS
R = 320
NPAD = NW * R
CAP = 16384
K = 128
ACCR = 336
DUMMY = 320
CE = 4000

_mesh = lambda: plsc.VectorSubcoreMesh(core_axis_name="c", subcore_axis_name="s")

_f32 = jnp.float32
_i32 = jnp.int32


def _wid():
    return lax.axis_index("s") * NC + lax.axis_index("c")


def _partition_call(src, dst):
    @functools.partial(
        pl.kernel,
        mesh=_mesh(),
        compiler_params=pltpu.CompilerParams(needs_layout_passes=False),
        out_type=(
            jax.ShapeDtypeStruct((NPAD,), _f32),
            jax.ShapeDtypeStruct((NW, CAP), _i32),
            jax.ShapeDtypeStruct((NW, CAP), _i32),
            jax.ShapeDtypeStruct((NW, L), _i32),
        ),
        scratch_types=[
            pltpu.VMEM((CE,), _i32),
            pltpu.VMEM((CE,), _i32),
            pltpu.VMEM((CE,), _i32),
            pltpu.VMEM((CE,), _i32),
            pltpu.VMEM((ACCR,), _f32),
            pltpu.VMEM((CAP,), _i32),
            pltpu.VMEM((CAP,), _i32),
            pltpu.VMEM((L,), _i32),
            pltpu.SemaphoreType.DMA,
            pltpu.SemaphoreType.DMA,
        ],
    )
    def p1(src_hbm, dst_hbm, deg_hbm, srcl_hbm, dlocl_hbm, cnt_hbm,
           sbuf0, dbuf0, sbuf1, dbuf1, dega, srca, dloca, cntv, sem0, sem1):
        iota = lax.iota(_i32, L)
        w = _wid()
        base = w * R
        nvalid = jnp.minimum(R, N - base)

        for i in range(ACCR // L):
            dega[pl.ds(i * L, L)] = jnp.zeros((L,), _f32)

        def issue(ci, sb, db, sem):
            pltpu.make_async_copy(src_hbm.at[pl.ds(ci * CE, CE)], sb, sem).start()
            pltpu.make_async_copy(dst_hbm.at[pl.ds(ci * CE, CE)], db, sem).start()

        def waitch(sb, db, sem):
            pltpu.make_async_copy(src_hbm.at[pl.ds(0, CE)], sb, sem).wait()
            pltpu.make_async_copy(dst_hbm.at[pl.ds(0, CE)], db, sem).wait()

        def scan_chunk(sb, db, offv):
            def grp(gi, offv):
                s16 = sb[pl.ds(gi * L, L)]
                d16 = db[pl.ds(gi * L, L)]
                dl = d16 - base
                m = (dl >= 0) & (dl < nvalid)
                dls = jnp.where(m, dl, DUMMY)
                plsc.addupdate_scatter(dega, [dls], jnp.where(m, 1.0, 0.0))
                cm = plsc.cumsum(m.astype(_i32))
                pos = jnp.where(m, offv + cm - 1, CAP - L + iota)
                plsc.store_scatter(srca, [pos], s16)
                plsc.store_scatter(dloca, [pos], dls)
                pcv = plsc.all_reduce_population_count(m)
                return jnp.minimum(offv + pcv, CAP - 1024)

            return lax.fori_loop(0, CE // L, grp, offv)

        NCH = E // CE
        issue(0, sbuf0, dbuf0, sem0)

        def half(h, offv):
            i1 = 2 * h + 1
            issue(i1, sbuf1, dbuf1, sem1)
            waitch(sbuf0, dbuf0, sem0)
            offv = scan_chunk(sbuf0, dbuf0, offv)

            @pl.when(i1 + 1 < NCH)
            def _():
                issue(i1 + 1, sbuf0, dbuf0, sem0)

            waitch(sbuf1, dbuf1, sem1)
            offv = scan_chunk(sbuf1, dbuf1, offv)
            return offv

        offv = lax.fori_loop(0, NCH // 2, half, jnp.zeros((L,), _i32))
        off = jnp.max(offv)

        def slgrp(j, off):
            idxv = off + iota
            plsc.store_scatter(srca, [idxv], base + j * L + iota)
            plsc.store_scatter(dloca, [idxv], j * L + iota)
            cur = plsc.load_gather(dega, [j * L + iota])
            plsc.store_scatter(dega, [j * L + iota], cur + 1.0)
            return off + L

        off = lax.fori_loop(0, nvalid // L, slgrp, off)

        target = ((off + K - 1) // K) * K
        for i in range(K // L):
            idxv = off + i * L + iota
            idxv = jnp.where(idxv < target, idxv, CAP - L + iota)
            plsc.store_scatter(srca, [idxv], jnp.zeros((L,), _i32))
            plsc.store_scatter(dloca, [idxv], jnp.full((L,), DUMMY, _i32))

        cntv[...] = lax.broadcast(target // K, (L,))
        pltpu.sync_copy(cntv, cnt_hbm.at[w])
        pltpu.sync_copy(dega.at[pl.ds(0, R)], deg_hbm.at[pl.ds(base, R)])
        pltpu.sync_copy(srca, srcl_hbm.at[w])
        pltpu.sync_copy(dloca, dlocl_hbm.at[w])

    return p1(src, dst)


@functools.lru_cache(maxsize=None)
def _make_agg(W):
    @functools.partial(
        pl.kernel,
        mesh=_mesh(),
        compiler_params=pltpu.CompilerParams(needs_layout_passes=False),
        out_type=jax.ShapeDtypeStruct((NPAD, W), _f32),
        scratch_types=[
            pltpu.VMEM((CAP,), _i32),
            pltpu.VMEM((CAP,), _i32),
            pltpu.VMEM((L,), _i32),
            pltpu.VMEM((ACCR, W), _f32),
            pltpu.VMEM((K, W), _f32),
            pltpu.VMEM((K, W), _f32),
            pltpu.SemaphoreType.DMA,
            pltpu.SemaphoreType.DMA,
        ],
    )
    def agg(x_hbm, srcl_hbm, dlocl_hbm, cnt_hbm, s_hbm,
            srca, dloca, cntv, acc, rows0, rows1, sem0, sem1):
        iota = lax.iota(_i32, L)
        w = _wid()
        base = w * R
        pltpu.sync_copy(cnt_hbm.at[w], cntv)
        nc = jnp.max(cntv[...])
        pltpu.sync_copy(srcl_hbm.at[w], srca)
        pltpu.sync_copy(dlocl_hbm.at[w], dloca)

        def zrow(r, _):
            for j in range(W // L):
                acc[r, pl.ds(j * L, L)] = jnp.zeros((L,), _f32)
            return 0

        lax.fori_loop(0, ACCR, zrow, 0)

        def issue(i, rows, sem):
            pltpu.make_async_copy(
                x_hbm.at[srca.at[pl.ds(i * K, K)]], rows, sem).start()

        def wait(rows, sem):
            pltpu.make_async_copy(
                x_hbm.at[srca.at[pl.ds(0, K)]], rows, sem).wait()

        def process(i, rows):
            def grp(g, _):
                dl16 = dloca[pl.ds(i * K + g * L, L)]
                for j in range(0, L, 2):
                    rsp0 = dl16.at[lax.broadcast(j, (L,))].get(
                        mode="promise_in_bounds")
                    rsp1 = dl16.at[lax.broadcast(j + 1, (L,))].get(
                        mode="promise_in_bounds")
                    e0 = g * L + j
                    e1 = e0 + 1
                    xs0 = [rows[e0, pl.ds(c * L, L)] for c in range(W // L)]
                    xs1 = [rows[e1, pl.ds(c * L, L)] for c in range(W // L)]
                    for c in range(W // L):
                        plsc.addupdate_scatter(acc, [rsp0, c * L + iota], xs0[c])
                    for c in range(W // L):
                        plsc.addupdate_scatter(acc, [rsp1, c * L + iota], xs1[c])
                return 0

            lax.fori_loop(0, K // L, grp, 0)

        issue(0, rows0, sem0)

        def half(h, _):
            i0 = 2 * h
            i1 = 2 * h + 1

            @pl.when(i1 < nc)
            def _():
                issue(i1, rows1, sem1)

            wait(rows0, sem0)
            process(i0, rows0)

            @pl.when(i1 < nc)
            def _():
                @pl.when(i1 + 1 < nc)
                def _():
                    issue(i1 + 1, rows0, sem0)

                wait(rows1, sem1)
                process(i1, rows1)

            return 0

        lax.fori_loop(0, (nc + 1) // 2, half, 0)
        pltpu.sync_copy(acc.at[pl.ds(0, R)], s_hbm.at[pl.ds(base, R)])

    return agg


def _rows(i, _=None):
    return (i, 0)


def _bcast(i, _=None):
    return (0, 0)


BLK = 512


def _tc_call(body, ins, blockable, out_widths):
    in_specs = [
        pl.BlockSpec((BLK, a.shape[1]), _rows) if b
        else pl.BlockSpec(a.shape, _bcast)
        for a, b in zip(ins, blockable)
    ]
    out_shape = tuple(jax.ShapeDtypeStruct((NPAD, wd), _f32) for wd in out_widths)
    out_specs = tuple(pl.BlockSpec((BLK, wd), _rows) for wd in out_widths)
    outs = pl.pallas_call(
        body,
        grid=(NPAD // BLK,),
        in_specs=in_specs,
        out_specs=out_specs,
        out_shape=out_shape,
    )(*ins)
    return outs


def _mm(a, b):
    return jnp.dot(a, b, preferred_element_type=_f32)


def kernel(feature, condition, edge_index,
           enc_f2h_W, enc_f2h_b, enc_c2h_W, enc_c2h_b, enc_h2h_W, enc_h2h_b,
           enc_mean_W, enc_mean_b, enc_logvar_W, enc_logvar_b,
           dec_z2h_W, dec_z2h_b, dec_c2h_W, dec_c2h_b, dec_h2h_W, dec_h2h_b,
           dec_out_W, dec_out_b):
    pad = NPAD - N
    fpad = jnp.pad(feature, ((0, pad), (0, 0)))
    cpad = jnp.pad(condition, ((0, pad), (0, 0)))
    noise = jax.random.normal(jax.random.key(1), (N, 64), _f32)
    npad_ = jnp.pad(noise, ((0, pad), (0, 0)))

    Whh1, Whh2 = enc_h2h_W[:128], enc_h2h_W[128:]
    Wdhh1, Wdhh2 = dec_h2h_W[:128], dec_h2h_W[128:]
    Wmlv = jnp.concatenate([enc_mean_W, enc_logvar_W], axis=1)
    bmlv = jnp.concatenate([enc_mean_b, enc_logvar_b]).reshape(1, 128)
    bf = enc_f2h_b.reshape(1, -1)
    bc = enc_c2h_b.reshape(1, -1)
    bh = enc_h2h_b.reshape(1, -1)
    bz = dec_z2h_b.reshape(1, -1)
    bdc = dec_c2h_b.reshape(1, -1)
    bdh = dec_h2h_b.reshape(1, -1)
    bout = dec_out_b.reshape(1, -1)

    deg, srcl, dlocl, cnt = _partition_call(edge_index[0], edge_index[1])
    degc = deg.reshape(NPAD, 1)

    agg128 = _make_agg(128)

    def tc0(deg_r, f_r, c_r, dinv_o, fs_o, cs_o):
        dv = lax.rsqrt(jnp.maximum(deg_r[...], 1.0))
        dinv_o[...] = dv
        fs_o[...] = f_r[...] * dv
        cs_o[...] = c_r[...] * dv

    dinv, fs, cs = _tc_call(tc0, [degc, fpad, cpad], [True] * 3, [1, 128, 128])

    s_f = agg128(fs, srcl, dlocl, cnt)
    s_c = agg128(cs, srcl, dlocl, cnt)

    def tc1(sf_r, sc_r, dv_r, wf, bf_r, wc, bc_r, wdc, bdc_r, whh1, whh2,
            wdhh2, ts_o, t2a_o):
        dv = dv_r[...]
        f2h = jnp.tanh(_mm(dv * sf_r[...], wf[...]) + bf_r[...])
        cpre = dv * sc_r[...]
        c2h = jnp.tanh(_mm(cpre, wc[...]) + bc_r[...])
        dc2h = jnp.tanh(_mm(cpre, wdc[...]) + bdc_r[...])
        ts_o[...] = dv * (_mm(f2h, whh1[...]) + _mm(c2h, whh2[...]))
        t2a_o[...] = _mm(dc2h, wdhh2[...])

    ts, t2a = _tc_call(
        tc1,
        [s_f, s_c, dinv, enc_f2h_W, bf, enc_c2h_W, bc, dec_c2h_W, bdc,
         Whh1, Whh2, Wdhh2],
        [True, True, True] + [False] * 9,
        [128, 128])

    s_t = agg128(ts, srcl, dlocl, cnt)

    def tc2(st_r, dv_r, bh_r, wmlv, ms_o):
        dv = dv_r[...]
        h = jnp.tanh(dv * st_r[...] + bh_r[...])
        ms_o[...] = dv * _mm(h, wmlv[...])

    (ms,) = _tc_call(tc2, [s_t, dinv, bh, Wmlv],
                     [True, True, False, False], [128])

    s_m = agg128(ms, srcl, dlocl, cnt)

    def tc3(sm_r, dv_r, bmlv_r, nz_r, mean_o, logvar_o, z_o, zs_o):
        dv = dv_r[...]
        mlv = dv * sm_r[...] + bmlv_r[...]
        mean = mlv[:, :64]
        logvar = mlv[:, 64:]
        z = nz_r[...] * jnp.exp(0.5 * logvar) + mean
        mean_o[...] = mean
        logvar_o[...] = logvar
        z_o[...] = z
        zs_o[...] = dv * z

    mean, logvar, z, zs = _tc_call(
        tc3, [s_m, dinv, bmlv, npad_],
        [True, True, False, True], [64, 64, 64, 64])

    zs128 = jnp.pad(zs, ((0, 0), (0, 64)))
    s_z = agg128(zs128, srcl, dlocl, cnt)

    def tc4(sz_r, dv_r, wz, bz_r, wdhh1, t2a_r, t2s_o):
        dv = dv_r[...]
        z2h = jnp.tanh(_mm(dv * sz_r[..., :64], wz[...]) + bz_r[...])
        t2s_o[...] = dv * (_mm(z2h, wdhh1[...]) + t2a_r[...])

    (t2s,) = _tc_call(tc4, [s_z, dinv, dec_z2h_W, bz, Wdhh1, t2a],
                      [True, True, False, False, False, True], [128])

    s_t2 = agg128(t2s, srcl, dlocl, cnt)

    def tc5(st2_r, dv_r, bdh_r, wout, t3s_o):
        dv = dv_r[...]
        dh = jnp.tanh(dv * st2_r[...] + bdh_r[...])
        t3s_o[...] = dv * _mm(dh, wout[...])

    (t3s,) = _tc_call(tc5, [s_t2, dinv, bdh, dec_out_W],
                      [True, True, False, False], [128])

    s_o = agg128(t3s, srcl, dlocl, cnt)

    def tc6(so_r, dv_r, bout_r, out_o):
        out_o[...] = dv_r[...] * so_r[...] + bout_r[...]

    (outp,) = _tc_call(tc6, [s_o, dinv, bout], [True, True, False], [128])

    return (z[:N], mean[:N], logvar[:N], outp[:N])


# speedup vs baseline: 1.0733x; 1.0733x over previous
import functools

import jax
import jax.numpy as jnp
from jax import lax
from jax.experimental import pallas as pl
from jax.experimental.pallas import tpu as pltpu
from jax.experimental.pallas import tpu_sc as plsc

N = 10000
E = 320000
NC, NS, L = 2, 16, 16
NW = NC * NS
R = 320
NPAD = NW * R
CAP = 16384
K = 128
ACCR = 336
DUMMY = 320
CE = 4000

_mesh = lambda: plsc.VectorSubcoreMesh(core_axis_name="c", subcore_axis_name="s")

_f32 = jnp.float32
_i32 = jnp.int32


def _wid():
    return lax.axis_index("s") * NC + lax.axis_index("c")


def _partition_call(src, dst):
    @functools.partial(
        pl.kernel,
        mesh=_mesh(),
        compiler_params=pltpu.CompilerParams(needs_layout_passes=False),
        out_type=(
            jax.ShapeDtypeStruct((NPAD,), _f32),
            jax.ShapeDtypeStruct((NW, CAP), _i32),
            jax.ShapeDtypeStruct((NW, CAP), _i32),
            jax.ShapeDtypeStruct((NW, L), _i32),
        ),
        scratch_types=[
            pltpu.VMEM((CE,), _i32),
            pltpu.VMEM((CE,), _i32),
            pltpu.VMEM((CE,), _i32),
            pltpu.VMEM((CE,), _i32),
            pltpu.VMEM((ACCR,), _f32),
            pltpu.VMEM((CAP,), _i32),
            pltpu.VMEM((CAP,), _i32),
            pltpu.VMEM((L,), _i32),
            pltpu.SemaphoreType.DMA,
            pltpu.SemaphoreType.DMA,
        ],
    )
    def p1(src_hbm, dst_hbm, deg_hbm, srcl_hbm, dlocl_hbm, cnt_hbm,
           sbuf0, dbuf0, sbuf1, dbuf1, dega, srca, dloca, cntv, sem0, sem1):
        iota = lax.iota(_i32, L)
        w = _wid()
        base = w * R
        nvalid = jnp.minimum(R, N - base)

        for i in range(ACCR // L):
            dega[pl.ds(i * L, L)] = jnp.zeros((L,), _f32)

        def issue(ci, sb, db, sem):
            pltpu.make_async_copy(src_hbm.at[pl.ds(ci * CE, CE)], sb, sem).start()
            pltpu.make_async_copy(dst_hbm.at[pl.ds(ci * CE, CE)], db, sem).start()

        def waitch(sb, db, sem):
            pltpu.make_async_copy(src_hbm.at[pl.ds(0, CE)], sb, sem).wait()
            pltpu.make_async_copy(dst_hbm.at[pl.ds(0, CE)], db, sem).wait()

        def scan_chunk(sb, db, offv):
            def grp(gi, offv):
                s16 = sb[pl.ds(gi * L, L)]
                d16 = db[pl.ds(gi * L, L)]
                dl = d16 - base
                m = (dl >= 0) & (dl < nvalid)
                dls = jnp.where(m, dl, DUMMY)
                plsc.addupdate_scatter(dega, [dls], jnp.where(m, 1.0, 0.0))
                cm = plsc.cumsum(m.astype(_i32))
                pos = jnp.where(m, offv + cm - 1, CAP - L + iota)
                plsc.store_scatter(srca, [pos], s16)
                plsc.store_scatter(dloca, [pos], dls)
                pcv = plsc.all_reduce_population_count(m)
                return jnp.minimum(offv + pcv, CAP - 1024)

            return lax.fori_loop(0, CE // L, grp, offv)

        NCH = E // CE
        issue(0, sbuf0, dbuf0, sem0)

        def half(h, offv):
            i1 = 2 * h + 1
            issue(i1, sbuf1, dbuf1, sem1)
            waitch(sbuf0, dbuf0, sem0)
            offv = scan_chunk(sbuf0, dbuf0, offv)

            @pl.when(i1 + 1 < NCH)
            def _():
                issue(i1 + 1, sbuf0, dbuf0, sem0)

            waitch(sbuf1, dbuf1, sem1)
            offv = scan_chunk(sbuf1, dbuf1, offv)
            return offv

        offv = lax.fori_loop(0, NCH // 2, half, jnp.zeros((L,), _i32))
        off = jnp.max(offv)

        def slgrp(j, off):
            idxv = off + iota
            plsc.store_scatter(srca, [idxv], base + j * L + iota)
            plsc.store_scatter(dloca, [idxv], j * L + iota)
            cur = plsc.load_gather(dega, [j * L + iota])
            plsc.store_scatter(dega, [j * L + iota], cur + 1.0)
            return off + L

        off = lax.fori_loop(0, nvalid // L, slgrp, off)

        target = ((off + K - 1) // K) * K
        for i in range(K // L):
            idxv = off + i * L + iota
            idxv = jnp.where(idxv < target, idxv, CAP - L + iota)
            plsc.store_scatter(srca, [idxv], jnp.zeros((L,), _i32))
            plsc.store_scatter(dloca, [idxv], jnp.full((L,), DUMMY, _i32))

        cntv[...] = lax.broadcast(target // K, (L,))
        pltpu.sync_copy(cntv, cnt_hbm.at[w])
        pltpu.sync_copy(dega.at[pl.ds(0, R)], deg_hbm.at[pl.ds(base, R)])
        pltpu.sync_copy(srca, srcl_hbm.at[w])
        pltpu.sync_copy(dloca, dlocl_hbm.at[w])

    return p1(src, dst)


@functools.lru_cache(maxsize=None)
def _make_agg(W):
    @functools.partial(
        pl.kernel,
        mesh=_mesh(),
        compiler_params=pltpu.CompilerParams(needs_layout_passes=False),
        out_type=jax.ShapeDtypeStruct((NPAD, W), _f32),
        scratch_types=[
            pltpu.VMEM((CAP,), _i32),
            pltpu.VMEM((CAP,), _i32),
            pltpu.VMEM((L,), _i32),
            pltpu.VMEM((ACCR, W), _f32),
            pltpu.VMEM((K, W), _f32),
            pltpu.VMEM((K, W), _f32),
            pltpu.SemaphoreType.DMA,
            pltpu.SemaphoreType.DMA,
        ],
    )
    def agg(x_hbm, srcl_hbm, dlocl_hbm, cnt_hbm, s_hbm,
            srca, dloca, cntv, acc, rows0, rows1, sem0, sem1):
        iota = lax.iota(_i32, L)
        w = _wid()
        base = w * R
        pltpu.sync_copy(cnt_hbm.at[w], cntv)
        nc = jnp.max(cntv[...])
        pltpu.sync_copy(srcl_hbm.at[w], srca)
        pltpu.sync_copy(dlocl_hbm.at[w], dloca)

        def zrow(r, _):
            for j in range(W // L):
                acc[r, pl.ds(j * L, L)] = jnp.zeros((L,), _f32)
            return 0

        lax.fori_loop(0, ACCR, zrow, 0)

        def issue(i, rows, sem):
            pltpu.make_async_copy(
                x_hbm.at[srca.at[pl.ds(i * K, K)]], rows, sem).start()

        def wait(rows, sem):
            pltpu.make_async_copy(
                x_hbm.at[srca.at[pl.ds(0, K)]], rows, sem).wait()

        def process(i, rows):
            def grp(g, _):
                dl16 = dloca[pl.ds(i * K + g * L, L)]
                for j in range(0, L, 2):
                    rsp0 = dl16.at[lax.broadcast(j, (L,))].get(
                        mode="promise_in_bounds")
                    rsp1 = dl16.at[lax.broadcast(j + 1, (L,))].get(
                        mode="promise_in_bounds")
                    e0 = g * L + j
                    e1 = e0 + 1
                    xs0 = [rows[e0, pl.ds(c * L, L)] for c in range(W // L)]
                    xs1 = [rows[e1, pl.ds(c * L, L)] for c in range(W // L)]
                    for c in range(W // L):
                        plsc.addupdate_scatter(acc, [rsp0, c * L + iota], xs0[c])
                    for c in range(W // L):
                        plsc.addupdate_scatter(acc, [rsp1, c * L + iota], xs1[c])
                return 0

            lax.fori_loop(0, K // L, grp, 0)

        issue(0, rows0, sem0)

        def half(h, _):
            i0 = 2 * h
            i1 = 2 * h + 1

            @pl.when(i1 < nc)
            def _():
                issue(i1, rows1, sem1)

            wait(rows0, sem0)
            process(i0, rows0)

            @pl.when(i1 < nc)
            def _():
                @pl.when(i1 + 1 < nc)
                def _():
                    issue(i1 + 1, rows0, sem0)

                wait(rows1, sem1)
                process(i1, rows1)

            return 0

        lax.fori_loop(0, (nc + 1) // 2, half, 0)
        pltpu.sync_copy(acc.at[pl.ds(0, R)], s_hbm.at[pl.ds(base, R)])

    return agg


K2 = 64


def _dual_agg_call(pk, srcl, dlocl, cnt):
    W = 128

    @functools.partial(
        pl.kernel,
        mesh=_mesh(),
        compiler_params=pltpu.CompilerParams(needs_layout_passes=False),
        out_type=(jax.ShapeDtypeStruct((NPAD, W), _f32),
                  jax.ShapeDtypeStruct((NPAD, W), _f32)),
        scratch_types=[
            pltpu.VMEM((CAP,), _i32),
            pltpu.VMEM((K2,), _i32),
            pltpu.VMEM((K2,), _i32),
            pltpu.VMEM((L,), _i32),
            pltpu.VMEM((ACCR, W), _f32),
            pltpu.VMEM((ACCR, W), _f32),
            pltpu.VMEM((K2, W), _i32),
            pltpu.VMEM((K2, W), _i32),
            pltpu.SemaphoreType.DMA,
            pltpu.SemaphoreType.DMA,
        ])
    def agg2(pk_hbm, srcl_hbm, dlocl_hbm, cnt_hbm, sf_hbm, sc_hbm,
             srca, dv0, dv1, cntv, accF, accC, rows0, rows1, sem0, sem1):
        iota = lax.iota(_i32, L)
        w = _wid()
        base = w * R
        pltpu.sync_copy(cnt_hbm.at[w], cntv)
        nc2 = jnp.max(cntv[...]) * 2
        pltpu.sync_copy(srcl_hbm.at[w], srca)

        def zrow(r, _):
            for j in range(W // L):
                accF[r, pl.ds(j * L, L)] = jnp.zeros((L,), _f32)
                accC[r, pl.ds(j * L, L)] = jnp.zeros((L,), _f32)
            return 0
        lax.fori_loop(0, ACCR, zrow, 0)

        def issue(i, rows, dv, sem):
            pltpu.make_async_copy(
                pk_hbm.at[srca.at[pl.ds(i * K2, K2)]], rows, sem).start()
            pltpu.make_async_copy(
                dlocl_hbm.at[w, pl.ds(i * K2, K2)], dv, sem).start()

        def wait(rows, dv, sem):
            pltpu.make_async_copy(
                pk_hbm.at[srca.at[pl.ds(0, K2)]], rows, sem).wait()
            pltpu.make_async_copy(
                dlocl_hbm.at[w, pl.ds(0, K2)], dv, sem).wait()

        def process(rows, dv):
            def grp(g, _):
                dl16 = dv[pl.ds(g * L, L)]
                for j in range(0, L, 2):
                    r0 = dl16.at[lax.broadcast(j, (L,))].get(
                        mode="promise_in_bounds")
                    r1 = dl16.at[lax.broadcast(j + 1, (L,))].get(
                        mode="promise_in_bounds")
                    e0 = g * L + j
                    e1 = e0 + 1

                    def halves(e):
                        out = []
                        for c in range(W // L):
                            v = rows[e, pl.ds(c * L, L)]
                            vb = plsc.bitcast(v, jnp.bfloat16)
                            out.append(plsc.unpack(
                                vb, format=plsc.PackFormat.INTERLEAVED))
                        return out

                    h0 = halves(e0)
                    h1 = halves(e1)
                    for c in range(W // L):
                        plsc.addupdate_scatter(
                            accF, [r0, c * L + iota], h0[c][0])
                        plsc.addupdate_scatter(
                            accC, [r0, c * L + iota], h0[c][1])
                    for c in range(W // L):
                        plsc.addupdate_scatter(
                            accF, [r1, c * L + iota], h1[c][0])
                        plsc.addupdate_scatter(
                            accC, [r1, c * L + iota], h1[c][1])
                return 0
            lax.fori_loop(0, K2 // L, grp, 0)

        issue(0, rows0, dv0, sem0)

        def half(h, _):
            i0, i1 = 2 * h, 2 * h + 1

            @pl.when(i1 < nc2)
            def _():
                issue(i1, rows1, dv1, sem1)
            wait(rows0, dv0, sem0)
            process(rows0, dv0)

            @pl.when(i1 < nc2)
            def _():
                @pl.when(i1 + 1 < nc2)
                def _():
                    issue(i1 + 1, rows0, dv0, sem0)
                wait(rows1, dv1, sem1)
                process(rows1, dv1)
            return 0

        lax.fori_loop(0, (nc2 + 1) // 2, half, 0)
        pltpu.sync_copy(accF.at[pl.ds(0, R)], sf_hbm.at[pl.ds(base, R)])
        pltpu.sync_copy(accC.at[pl.ds(0, R)], sc_hbm.at[pl.ds(base, R)])

    return agg2(pk, srcl, dlocl, cnt)


def _rows(i, _=None):
    return (i, 0)


def _bcast(i, _=None):
    return (0, 0)


BLK = 512


def _tc_call(body, ins, blockable, out_widths):
    in_specs = [
        pl.BlockSpec((BLK, a.shape[1]), _rows) if b
        else pl.BlockSpec(a.shape, _bcast)
        for a, b in zip(ins, blockable)
    ]
    out_widths = [w if isinstance(w, tuple) else (w, _f32) for w in out_widths]
    out_shape = tuple(
        jax.ShapeDtypeStruct((NPAD, wd), dt) for wd, dt in out_widths)
    out_specs = tuple(pl.BlockSpec((BLK, wd), _rows) for wd, _ in out_widths)
    outs = pl.pallas_call(
        body,
        grid=(NPAD // BLK,),
        in_specs=in_specs,
        out_specs=out_specs,
        out_shape=out_shape,
    )(*ins)
    return outs


def _mm(a, b):
    return jnp.dot(a, b, preferred_element_type=_f32)


def kernel(feature, condition, edge_index,
           enc_f2h_W, enc_f2h_b, enc_c2h_W, enc_c2h_b, enc_h2h_W, enc_h2h_b,
           enc_mean_W, enc_mean_b, enc_logvar_W, enc_logvar_b,
           dec_z2h_W, dec_z2h_b, dec_c2h_W, dec_c2h_b, dec_h2h_W, dec_h2h_b,
           dec_out_W, dec_out_b):
    pad = NPAD - N
    fpad = jnp.pad(feature, ((0, pad), (0, 0)))
    cpad = jnp.pad(condition, ((0, pad), (0, 0)))
    noise = jax.random.normal(jax.random.key(1), (N, 64), _f32)
    npad_ = jnp.pad(noise, ((0, pad), (0, 0)))

    Whh1, Whh2 = enc_h2h_W[:128], enc_h2h_W[128:]
    Wdhh1, Wdhh2 = dec_h2h_W[:128], dec_h2h_W[128:]
    Wmlv = jnp.concatenate([enc_mean_W, enc_logvar_W], axis=1)
    bmlv = jnp.concatenate([enc_mean_b, enc_logvar_b]).reshape(1, 128)
    bf = enc_f2h_b.reshape(1, -1)
    bc = enc_c2h_b.reshape(1, -1)
    bh = enc_h2h_b.reshape(1, -1)
    bz = dec_z2h_b.reshape(1, -1)
    bdc = dec_c2h_b.reshape(1, -1)
    bdh = dec_h2h_b.reshape(1, -1)
    bout = dec_out_b.reshape(1, -1)

    deg, srcl, dlocl, cnt = _partition_call(edge_index[0], edge_index[1])
    degc = deg.reshape(NPAD, 1)

    agg128 = _make_agg(128)

    def tc0(deg_r, f_r, c_r, dinv_o, pk_o):
        dv = lax.rsqrt(jnp.maximum(deg_r[...], 1.0))
        dinv_o[...] = dv
        fb = lax.bitcast_convert_type(
            (f_r[...] * dv).astype(jnp.bfloat16), jnp.uint16).astype(jnp.uint32)
        cb = lax.bitcast_convert_type(
            (c_r[...] * dv).astype(jnp.bfloat16), jnp.uint16).astype(jnp.uint32)
        pk_o[...] = lax.bitcast_convert_type(fb | (cb << 16), jnp.int32)

    dinv, pk = _tc_call(tc0, [degc, fpad, cpad], [True] * 3,
                        [1, (128, jnp.int32)])

    s_f, s_c = _dual_agg_call(pk, srcl, dlocl, cnt)

    def tc1(sf_r, sc_r, dv_r, wf, bf_r, wc, bc_r, wdc, bdc_r, whh1, whh2,
            wdhh2, ts_o, t2a_o):
        dv = dv_r[...]
        f2h = jnp.tanh(_mm(dv * sf_r[...], wf[...]) + bf_r[...])
        cpre = dv * sc_r[...]
        c2h = jnp.tanh(_mm(cpre, wc[...]) + bc_r[...])
        dc2h = jnp.tanh(_mm(cpre, wdc[...]) + bdc_r[...])
        ts_o[...] = dv * (_mm(f2h, whh1[...]) + _mm(c2h, whh2[...]))
        t2a_o[...] = _mm(dc2h, wdhh2[...])

    ts, t2a = _tc_call(
        tc1,
        [s_f, s_c, dinv, enc_f2h_W, bf, enc_c2h_W, bc, dec_c2h_W, bdc,
         Whh1, Whh2, Wdhh2],
        [True, True, True] + [False] * 9,
        [128, 128])

    s_t = agg128(ts, srcl, dlocl, cnt)

    def tc2(st_r, dv_r, bh_r, wmlv, ms_o):
        dv = dv_r[...]
        h = jnp.tanh(dv * st_r[...] + bh_r[...])
        ms_o[...] = dv * _mm(h, wmlv[...])

    (ms,) = _tc_call(tc2, [s_t, dinv, bh, Wmlv],
                     [True, True, False, False], [128])

    s_m = agg128(ms, srcl, dlocl, cnt)

    def tc3(sm_r, dv_r, bmlv_r, nz_r, mean_o, logvar_o, z_o, zs_o):
        dv = dv_r[...]
        mlv = dv * sm_r[...] + bmlv_r[...]
        mean = mlv[:, :64]
        logvar = mlv[:, 64:]
        z = nz_r[...] * jnp.exp(0.5 * logvar) + mean
        mean_o[...] = mean
        logvar_o[...] = logvar
        z_o[...] = z
        zs_o[...] = dv * z

    mean, logvar, z, zs = _tc_call(
        tc3, [s_m, dinv, bmlv, npad_],
        [True, True, False, True], [64, 64, 64, 64])

    zs128 = jnp.pad(zs, ((0, 0), (0, 64)))
    s_z = agg128(zs128, srcl, dlocl, cnt)

    def tc4(sz_r, dv_r, wz, bz_r, wdhh1, t2a_r, t2s_o):
        dv = dv_r[...]
        z2h = jnp.tanh(_mm(dv * sz_r[..., :64], wz[...]) + bz_r[...])
        t2s_o[...] = dv * (_mm(z2h, wdhh1[...]) + t2a_r[...])

    (t2s,) = _tc_call(tc4, [s_z, dinv, dec_z2h_W, bz, Wdhh1, t2a],
                      [True, True, False, False, False, True], [128])

    s_t2 = agg128(t2s, srcl, dlocl, cnt)

    def tc5(st2_r, dv_r, bdh_r, wout, t3s_o):
        dv = dv_r[...]
        dh = jnp.tanh(dv * st2_r[...] + bdh_r[...])
        t3s_o[...] = dv * _mm(dh, wout[...])

    (t3s,) = _tc_call(tc5, [s_t2, dinv, bdh, dec_out_W],
                      [True, True, False, False], [128])

    s_o = agg128(t3s, srcl, dlocl, cnt)

    def tc6(so_r, dv_r, bout_r, out_o):
        out_o[...] = dv_r[...] * so_r[...] + bout_r[...]

    (outp,) = _tc_call(tc6, [s_o, dinv, bout], [True, True, False], [128])

    return (z[:N], mean[:N], logvar[:N], outp[:N])


# speedup vs baseline: 1.1505x; 1.0719x over previous
import functools

import jax
import jax.numpy as jnp
from jax import lax
from jax.experimental import pallas as pl
from jax.experimental.pallas import tpu as pltpu
from jax.experimental.pallas import tpu_sc as plsc

N = 10000
E = 320000
NC, NS, L = 2, 16, 16
NW = NC * NS
R = 320
NPAD = NW * R
CAP = 16384
K = 128
ACCR = 336
DUMMY = 320
CE = 4000

_mesh = lambda: plsc.VectorSubcoreMesh(core_axis_name="c", subcore_axis_name="s")

_f32 = jnp.float32
_i32 = jnp.int32


def _wid():
    return lax.axis_index("s") * NC + lax.axis_index("c")


def _partition_call(src, dst):
    @functools.partial(
        pl.kernel,
        mesh=_mesh(),
        compiler_params=pltpu.CompilerParams(needs_layout_passes=False),
        out_type=(
            jax.ShapeDtypeStruct((NPAD,), _f32),
            jax.ShapeDtypeStruct((NW, CAP), _i32),
            jax.ShapeDtypeStruct((NW, CAP), _i32),
            jax.ShapeDtypeStruct((NW, L), _i32),
        ),
        scratch_types=[
            pltpu.VMEM((CE,), _i32),
            pltpu.VMEM((CE,), _i32),
            pltpu.VMEM((CE,), _i32),
            pltpu.VMEM((CE,), _i32),
            pltpu.VMEM((ACCR,), _f32),
            pltpu.VMEM((CAP,), _i32),
            pltpu.VMEM((CAP,), _i32),
            pltpu.VMEM((L,), _i32),
            pltpu.SemaphoreType.DMA,
            pltpu.SemaphoreType.DMA,
        ],
    )
    def p1(src_hbm, dst_hbm, deg_hbm, srcl_hbm, dlocl_hbm, cnt_hbm,
           sbuf0, dbuf0, sbuf1, dbuf1, dega, srca, dloca, cntv, sem0, sem1):
        iota = lax.iota(_i32, L)
        w = _wid()
        base = w * R
        nvalid = jnp.minimum(R, N - base)

        for i in range(ACCR // L):
            dega[pl.ds(i * L, L)] = jnp.zeros((L,), _f32)

        def issue(ci, sb, db, sem):
            pltpu.make_async_copy(src_hbm.at[pl.ds(ci * CE, CE)], sb, sem).start()
            pltpu.make_async_copy(dst_hbm.at[pl.ds(ci * CE, CE)], db, sem).start()

        def waitch(sb, db, sem):
            pltpu.make_async_copy(src_hbm.at[pl.ds(0, CE)], sb, sem).wait()
            pltpu.make_async_copy(dst_hbm.at[pl.ds(0, CE)], db, sem).wait()

        def scan_chunk(sb, db, offv):
            def grp(gi2, offv):
                g0 = gi2 * 2
                s16a = sb[pl.ds(g0 * L, L)]
                d16a = db[pl.ds(g0 * L, L)]
                s16b = sb[pl.ds((g0 + 1) * L, L)]
                d16b = db[pl.ds((g0 + 1) * L, L)]
                dla = d16a - base
                dlb = d16b - base
                ma = (dla >= 0) & (dla < nvalid)
                mb = (dlb >= 0) & (dlb < nvalid)
                dlsa = jnp.where(ma, dla, DUMMY)
                dlsb = jnp.where(mb, dlb, DUMMY)
                cma = plsc.cumsum(ma.astype(_i32))
                cmb = plsc.cumsum(mb.astype(_i32))
                pca = plsc.all_reduce_population_count(ma)
                pcb = plsc.all_reduce_population_count(mb)
                posa = jnp.where(ma, offv + cma - 1, CAP - L + iota)
                offv1 = offv + pca
                posb = jnp.where(mb, offv1 + cmb - 1, CAP - L + iota)
                plsc.addupdate_scatter(dega, [dlsa], jnp.where(ma, 1.0, 0.0))
                plsc.addupdate_scatter(dega, [dlsb], jnp.where(mb, 1.0, 0.0))
                plsc.store_scatter(srca, [posa], s16a)
                plsc.store_scatter(dloca, [posa], dlsa)
                plsc.store_scatter(srca, [posb], s16b)
                plsc.store_scatter(dloca, [posb], dlsb)
                return jnp.minimum(offv1 + pcb, CAP - 1024)

            return lax.fori_loop(0, CE // (2 * L), grp, offv)

        NCH = E // CE
        issue(0, sbuf0, dbuf0, sem0)

        def half(h, offv):
            i1 = 2 * h + 1
            issue(i1, sbuf1, dbuf1, sem1)
            waitch(sbuf0, dbuf0, sem0)
            offv = scan_chunk(sbuf0, dbuf0, offv)

            @pl.when(i1 + 1 < NCH)
            def _():
                issue(i1 + 1, sbuf0, dbuf0, sem0)

            waitch(sbuf1, dbuf1, sem1)
            offv = scan_chunk(sbuf1, dbuf1, offv)
            return offv

        offv = lax.fori_loop(0, NCH // 2, half, jnp.zeros((L,), _i32))
        off = jnp.max(offv)

        def slgrp(j, off):
            idxv = off + iota
            plsc.store_scatter(srca, [idxv], base + j * L + iota)
            plsc.store_scatter(dloca, [idxv], j * L + iota)
            cur = plsc.load_gather(dega, [j * L + iota])
            plsc.store_scatter(dega, [j * L + iota], cur + 1.0)
            return off + L

        off = lax.fori_loop(0, nvalid // L, slgrp, off)

        target = ((off + K - 1) // K) * K
        for i in range(K // L):
            idxv = off + i * L + iota
            idxv = jnp.where(idxv < target, idxv, CAP - L + iota)
            plsc.store_scatter(srca, [idxv], jnp.zeros((L,), _i32))
            plsc.store_scatter(dloca, [idxv], jnp.full((L,), DUMMY, _i32))

        cntv[...] = lax.broadcast(target // K, (L,))
        pltpu.sync_copy(cntv, cnt_hbm.at[w])
        pltpu.sync_copy(dega.at[pl.ds(0, R)], deg_hbm.at[pl.ds(base, R)])
        pltpu.sync_copy(srca, srcl_hbm.at[w])
        pltpu.sync_copy(dloca, dlocl_hbm.at[w])

    return p1(src, dst)


@functools.lru_cache(maxsize=None)
def _make_agg(W):
    @functools.partial(
        pl.kernel,
        mesh=_mesh(),
        compiler_params=pltpu.CompilerParams(needs_layout_passes=False),
        out_type=jax.ShapeDtypeStruct((NPAD, W), _f32),
        scratch_types=[
            pltpu.VMEM((CAP,), _i32),
            pltpu.VMEM((CAP,), _i32),
            pltpu.VMEM((L,), _i32),
            pltpu.VMEM((ACCR, W), _f32),
            pltpu.VMEM((K, W), _f32),
            pltpu.VMEM((K, W), _f32),
            pltpu.SemaphoreType.DMA,
            pltpu.SemaphoreType.DMA,
        ],
    )
    def agg(x_hbm, srcl_hbm, dlocl_hbm, cnt_hbm, s_hbm,
            srca, dloca, cntv, acc, rows0, rows1, sem0, sem1):
        iota = lax.iota(_i32, L)
        w = _wid()
        base = w * R
        pltpu.sync_copy(cnt_hbm.at[w], cntv)
        nc = jnp.max(cntv[...])
        pltpu.sync_copy(srcl_hbm.at[w], srca)
        pltpu.sync_copy(dlocl_hbm.at[w], dloca)

        def zrow(r, _):
            for j in range(W // L):
                acc[r, pl.ds(j * L, L)] = jnp.zeros((L,), _f32)
            return 0

        lax.fori_loop(0, ACCR, zrow, 0)

        def issue(i, rows, sem):
            pltpu.make_async_copy(
                x_hbm.at[srca.at[pl.ds(i * K, K)]], rows, sem).start()

        def wait(rows, sem):
            pltpu.make_async_copy(
                x_hbm.at[srca.at[pl.ds(0, K)]], rows, sem).wait()

        def process(i, rows):
            def grp(g, _):
                dl16 = dloca[pl.ds(i * K + g * L, L)]
                for j in range(0, L, 2):
                    rsp0 = dl16.at[lax.broadcast(j, (L,))].get(
                        mode="promise_in_bounds")
                    rsp1 = dl16.at[lax.broadcast(j + 1, (L,))].get(
                        mode="promise_in_bounds")
                    e0 = g * L + j
                    e1 = e0 + 1
                    xs0 = [rows[e0, pl.ds(c * L, L)] for c in range(W // L)]
                    xs1 = [rows[e1, pl.ds(c * L, L)] for c in range(W // L)]
                    for c in range(W // L):
                        plsc.addupdate_scatter(acc, [rsp0, c * L + iota], xs0[c])
                    for c in range(W // L):
                        plsc.addupdate_scatter(acc, [rsp1, c * L + iota], xs1[c])
                return 0

            lax.fori_loop(0, K // L, grp, 0)

        issue(0, rows0, sem0)

        def half(h, _):
            i0 = 2 * h
            i1 = 2 * h + 1

            @pl.when(i1 < nc)
            def _():
                issue(i1, rows1, sem1)

            wait(rows0, sem0)
            process(i0, rows0)

            @pl.when(i1 < nc)
            def _():
                @pl.when(i1 + 1 < nc)
                def _():
                    issue(i1 + 1, rows0, sem0)

                wait(rows1, sem1)
                process(i1, rows1)

            return 0

        lax.fori_loop(0, (nc + 1) // 2, half, 0)
        pltpu.sync_copy(acc.at[pl.ds(0, R)], s_hbm.at[pl.ds(base, R)])

    return agg


K2 = 64


def _dual_agg_call(pk, srcl, dlocl, cnt):
    W = 128

    @functools.partial(
        pl.kernel,
        mesh=_mesh(),
        compiler_params=pltpu.CompilerParams(needs_layout_passes=False),
        out_type=(jax.ShapeDtypeStruct((NPAD, W), _f32),
                  jax.ShapeDtypeStruct((NPAD, W), _f32)),
        scratch_types=[
            pltpu.VMEM((CAP,), _i32),
            pltpu.VMEM((K2,), _i32),
            pltpu.VMEM((K2,), _i32),
            pltpu.VMEM((L,), _i32),
            pltpu.VMEM((ACCR, W), _f32),
            pltpu.VMEM((ACCR, W), _f32),
            pltpu.VMEM((K2, W), _i32),
            pltpu.VMEM((K2, W), _i32),
            pltpu.SemaphoreType.DMA,
            pltpu.SemaphoreType.DMA,
        ])
    def agg2(pk_hbm, srcl_hbm, dlocl_hbm, cnt_hbm, sf_hbm, sc_hbm,
             srca, dv0, dv1, cntv, accF, accC, rows0, rows1, sem0, sem1):
        iota = lax.iota(_i32, L)
        w = _wid()
        base = w * R
        pltpu.sync_copy(cnt_hbm.at[w], cntv)
        nc2 = jnp.max(cntv[...]) * 2
        pltpu.sync_copy(srcl_hbm.at[w], srca)

        def zrow(r, _):
            for j in range(W // L):
                accF[r, pl.ds(j * L, L)] = jnp.zeros((L,), _f32)
                accC[r, pl.ds(j * L, L)] = jnp.zeros((L,), _f32)
            return 0
        lax.fori_loop(0, ACCR, zrow, 0)

        def issue(i, rows, dv, sem):
            pltpu.make_async_copy(
                pk_hbm.at[srca.at[pl.ds(i * K2, K2)]], rows, sem).start()
            pltpu.make_async_copy(
                dlocl_hbm.at[w, pl.ds(i * K2, K2)], dv, sem).start()

        def wait(rows, dv, sem):
            pltpu.make_async_copy(
                pk_hbm.at[srca.at[pl.ds(0, K2)]], rows, sem).wait()
            pltpu.make_async_copy(
                dlocl_hbm.at[w, pl.ds(0, K2)], dv, sem).wait()

        def process(rows, dv):
            def grp(g, _):
                dl16 = dv[pl.ds(g * L, L)]
                for j in range(0, L, 2):
                    r0 = dl16.at[lax.broadcast(j, (L,))].get(
                        mode="promise_in_bounds")
                    r1 = dl16.at[lax.broadcast(j + 1, (L,))].get(
                        mode="promise_in_bounds")
                    e0 = g * L + j
                    e1 = e0 + 1

                    def halves(e):
                        out = []
                        for c in range(W // L):
                            v = rows[e, pl.ds(c * L, L)]
                            vb = plsc.bitcast(v, jnp.bfloat16)
                            out.append(plsc.unpack(
                                vb, format=plsc.PackFormat.INTERLEAVED))
                        return out

                    h0 = halves(e0)
                    h1 = halves(e1)
                    for c in range(W // L):
                        plsc.addupdate_scatter(
                            accF, [r0, c * L + iota], h0[c][0])
                        plsc.addupdate_scatter(
                            accC, [r0, c * L + iota], h0[c][1])
                    for c in range(W // L):
                        plsc.addupdate_scatter(
                            accF, [r1, c * L + iota], h1[c][0])
                        plsc.addupdate_scatter(
                            accC, [r1, c * L + iota], h1[c][1])
                return 0
            lax.fori_loop(0, K2 // L, grp, 0)

        issue(0, rows0, dv0, sem0)

        def half(h, _):
            i0, i1 = 2 * h, 2 * h + 1

            @pl.when(i1 < nc2)
            def _():
                issue(i1, rows1, dv1, sem1)
            wait(rows0, dv0, sem0)
            process(rows0, dv0)

            @pl.when(i1 < nc2)
            def _():
                @pl.when(i1 + 1 < nc2)
                def _():
                    issue(i1 + 1, rows0, dv0, sem0)
                wait(rows1, dv1, sem1)
                process(rows1, dv1)
            return 0

        lax.fori_loop(0, (nc2 + 1) // 2, half, 0)
        pltpu.sync_copy(accF.at[pl.ds(0, R)], sf_hbm.at[pl.ds(base, R)])
        pltpu.sync_copy(accC.at[pl.ds(0, R)], sc_hbm.at[pl.ds(base, R)])

    return agg2(pk, srcl, dlocl, cnt)


def _rows(i, _=None):
    return (i, 0)


def _bcast(i, _=None):
    return (0, 0)


BLK = 512


def _tc_call(body, ins, blockable, out_widths):
    in_specs = [
        pl.BlockSpec((BLK, a.shape[1]), _rows) if b
        else pl.BlockSpec(a.shape, _bcast)
        for a, b in zip(ins, blockable)
    ]
    out_widths = [w if isinstance(w, tuple) else (w, _f32) for w in out_widths]
    out_shape = tuple(
        jax.ShapeDtypeStruct((NPAD, wd), dt) for wd, dt in out_widths)
    out_specs = tuple(pl.BlockSpec((BLK, wd), _rows) for wd, _ in out_widths)
    outs = pl.pallas_call(
        body,
        grid=(NPAD // BLK,),
        in_specs=in_specs,
        out_specs=out_specs,
        out_shape=out_shape,
    )(*ins)
    return outs


def _mm(a, b):
    return jnp.dot(a, b, preferred_element_type=_f32)


def kernel(feature, condition, edge_index,
           enc_f2h_W, enc_f2h_b, enc_c2h_W, enc_c2h_b, enc_h2h_W, enc_h2h_b,
           enc_mean_W, enc_mean_b, enc_logvar_W, enc_logvar_b,
           dec_z2h_W, dec_z2h_b, dec_c2h_W, dec_c2h_b, dec_h2h_W, dec_h2h_b,
           dec_out_W, dec_out_b):
    pad = NPAD - N
    fpad = jnp.pad(feature, ((0, pad), (0, 0)))
    cpad = jnp.pad(condition, ((0, pad), (0, 0)))
    noise = jax.random.normal(jax.random.key(1), (N, 64), _f32)
    npad_ = jnp.pad(noise, ((0, pad), (0, 0)))

    Whh1, Whh2 = enc_h2h_W[:128], enc_h2h_W[128:]
    Wdhh1, Wdhh2 = dec_h2h_W[:128], dec_h2h_W[128:]
    Wmlv = jnp.concatenate([enc_mean_W, enc_logvar_W], axis=1)
    bmlv = jnp.concatenate([enc_mean_b, enc_logvar_b]).reshape(1, 128)
    bf = enc_f2h_b.reshape(1, -1)
    bc = enc_c2h_b.reshape(1, -1)
    bh = enc_h2h_b.reshape(1, -1)
    bz = dec_z2h_b.reshape(1, -1)
    bdc = dec_c2h_b.reshape(1, -1)
    bdh = dec_h2h_b.reshape(1, -1)
    bout = dec_out_b.reshape(1, -1)

    deg, srcl, dlocl, cnt = _partition_call(edge_index[0], edge_index[1])
    degc = deg.reshape(NPAD, 1)

    agg128 = _make_agg(128)

    def tc0(deg_r, f_r, c_r, dinv_o, pk_o):
        dv = lax.rsqrt(jnp.maximum(deg_r[...], 1.0))
        dinv_o[...] = dv
        fb = lax.bitcast_convert_type(
            (f_r[...] * dv).astype(jnp.bfloat16), jnp.uint16).astype(jnp.uint32)
        cb = lax.bitcast_convert_type(
            (c_r[...] * dv).astype(jnp.bfloat16), jnp.uint16).astype(jnp.uint32)
        pk_o[...] = lax.bitcast_convert_type(fb | (cb << 16), jnp.int32)

    dinv, pk = _tc_call(tc0, [degc, fpad, cpad], [True] * 3,
                        [1, (128, jnp.int32)])

    s_f, s_c = _dual_agg_call(pk, srcl, dlocl, cnt)

    def tc1(sf_r, sc_r, dv_r, wf, bf_r, wc, bc_r, wdc, bdc_r, whh1, whh2,
            wdhh2, ts_o, t2a_o):
        dv = dv_r[...]
        f2h = jnp.tanh(_mm(dv * sf_r[...], wf[...]) + bf_r[...])
        cpre = dv * sc_r[...]
        c2h = jnp.tanh(_mm(cpre, wc[...]) + bc_r[...])
        dc2h = jnp.tanh(_mm(cpre, wdc[...]) + bdc_r[...])
        ts_o[...] = dv * (_mm(f2h, whh1[...]) + _mm(c2h, whh2[...]))
        t2a_o[...] = _mm(dc2h, wdhh2[...])

    ts, t2a = _tc_call(
        tc1,
        [s_f, s_c, dinv, enc_f2h_W, bf, enc_c2h_W, bc, dec_c2h_W, bdc,
         Whh1, Whh2, Wdhh2],
        [True, True, True] + [False] * 9,
        [128, 128])

    s_t = agg128(ts, srcl, dlocl, cnt)

    def tc2(st_r, dv_r, bh_r, wmlv, ms_o):
        dv = dv_r[...]
        h = jnp.tanh(dv * st_r[...] + bh_r[...])
        ms_o[...] = dv * _mm(h, wmlv[...])

    (ms,) = _tc_call(tc2, [s_t, dinv, bh, Wmlv],
                     [True, True, False, False], [128])

    s_m = agg128(ms, srcl, dlocl, cnt)

    def tc3(sm_r, dv_r, bmlv_r, nz_r, mean_o, logvar_o, z_o, zs_o):
        dv = dv_r[...]
        mlv = dv * sm_r[...] + bmlv_r[...]
        mean = mlv[:, :64]
        logvar = mlv[:, 64:]
        z = nz_r[...] * jnp.exp(0.5 * logvar) + mean
        mean_o[...] = mean
        logvar_o[...] = logvar
        z_o[...] = z
        zs_o[...] = dv * z

    mean, logvar, z, zs = _tc_call(
        tc3, [s_m, dinv, bmlv, npad_],
        [True, True, False, True], [64, 64, 64, 64])

    zs128 = jnp.pad(zs, ((0, 0), (0, 64)))
    s_z = agg128(zs128, srcl, dlocl, cnt)

    def tc4(sz_r, dv_r, wz, bz_r, wdhh1, t2a_r, t2s_o):
        dv = dv_r[...]
        z2h = jnp.tanh(_mm(dv * sz_r[..., :64], wz[...]) + bz_r[...])
        t2s_o[...] = dv * (_mm(z2h, wdhh1[...]) + t2a_r[...])

    (t2s,) = _tc_call(tc4, [s_z, dinv, dec_z2h_W, bz, Wdhh1, t2a],
                      [True, True, False, False, False, True], [128])

    s_t2 = agg128(t2s, srcl, dlocl, cnt)

    def tc5(st2_r, dv_r, bdh_r, wout, t3s_o):
        dv = dv_r[...]
        dh = jnp.tanh(dv * st2_r[...] + bdh_r[...])
        t3s_o[...] = dv * _mm(dh, wout[...])

    (t3s,) = _tc_call(tc5, [s_t2, dinv, bdh, dec_out_W],
                      [True, True, False, False], [128])

    s_o = agg128(t3s, srcl, dlocl, cnt)

    def tc6(so_r, dv_r, bout_r, out_o):
        out_o[...] = dv_r[...] * so_r[...] + bout_r[...]

    (outp,) = _tc_call(tc6, [s_o, dinv, bout], [True, True, False], [128])

    return (z[:N], mean[:N], logvar[:N], outp[:N])


# speedup vs baseline: 1.1769x; 1.0230x over previous
import functools

import jax
import jax.numpy as jnp
from jax import lax
from jax.experimental import pallas as pl
from jax.experimental.pallas import tpu as pltpu
from jax.experimental.pallas import tpu_sc as plsc

N = 10000
E = 320000
NC, NS, L = 2, 16, 16
NW = NC * NS
R = 320
NPAD = NW * R
CAP = 16384
K = 128
ACCR = 336
DUMMY = 320
CE = 4000

_mesh = lambda: plsc.VectorSubcoreMesh(core_axis_name="c", subcore_axis_name="s")

_f32 = jnp.float32
_i32 = jnp.int32


def _wid():
    return lax.axis_index("s") * NC + lax.axis_index("c")


def _partition_call(src, dst):
    @functools.partial(
        pl.kernel,
        mesh=_mesh(),
        compiler_params=pltpu.CompilerParams(needs_layout_passes=False),
        out_type=(
            jax.ShapeDtypeStruct((NPAD,), _f32),
            jax.ShapeDtypeStruct((NW, CAP), _i32),
            jax.ShapeDtypeStruct((NW, CAP), _i32),
            jax.ShapeDtypeStruct((NW, L), _i32),
        ),
        scratch_types=[
            pltpu.VMEM((CE,), _i32),
            pltpu.VMEM((CE,), _i32),
            pltpu.VMEM((CE,), _i32),
            pltpu.VMEM((CE,), _i32),
            pltpu.VMEM((ACCR,), _f32),
            pltpu.VMEM((CAP,), _i32),
            pltpu.VMEM((CAP,), _i32),
            pltpu.VMEM((L,), _i32),
            pltpu.SemaphoreType.DMA,
            pltpu.SemaphoreType.DMA,
        ],
    )
    def p1(src_hbm, dst_hbm, deg_hbm, srcl_hbm, dlocl_hbm, cnt_hbm,
           sbuf0, dbuf0, sbuf1, dbuf1, dega, srca, dloca, cntv, sem0, sem1):
        iota = lax.iota(_i32, L)
        w = _wid()
        base = w * R
        nvalid = jnp.minimum(R, N - base)

        for i in range(ACCR // L):
            dega[pl.ds(i * L, L)] = jnp.zeros((L,), _f32)

        def issue(ci, sb, db, sem):
            pltpu.make_async_copy(src_hbm.at[pl.ds(ci * CE, CE)], sb, sem).start()
            pltpu.make_async_copy(dst_hbm.at[pl.ds(ci * CE, CE)], db, sem).start()

        def waitch(sb, db, sem):
            pltpu.make_async_copy(src_hbm.at[pl.ds(0, CE)], sb, sem).wait()
            pltpu.make_async_copy(dst_hbm.at[pl.ds(0, CE)], db, sem).wait()

        def scan_chunk(sb, db, offv):
            def grp(gi2, offv):
                g0 = gi2 * 2
                s16a = sb[pl.ds(g0 * L, L)]
                d16a = db[pl.ds(g0 * L, L)]
                s16b = sb[pl.ds((g0 + 1) * L, L)]
                d16b = db[pl.ds((g0 + 1) * L, L)]
                dla = d16a - base
                dlb = d16b - base
                ma = (dla >= 0) & (dla < nvalid)
                mb = (dlb >= 0) & (dlb < nvalid)
                dlsa = jnp.where(ma, dla, DUMMY)
                dlsb = jnp.where(mb, dlb, DUMMY)
                cma = plsc.cumsum(ma.astype(_i32))
                cmb = plsc.cumsum(mb.astype(_i32))
                pca = plsc.all_reduce_population_count(ma)
                pcb = plsc.all_reduce_population_count(mb)
                posa = jnp.where(ma, offv + cma - 1, CAP - L + iota)
                offv1 = offv + pca
                posb = jnp.where(mb, offv1 + cmb - 1, CAP - L + iota)
                plsc.addupdate_scatter(dega, [dlsa], jnp.where(ma, 1.0, 0.0))
                plsc.addupdate_scatter(dega, [dlsb], jnp.where(mb, 1.0, 0.0))
                plsc.store_scatter(srca, [posa], s16a)
                plsc.store_scatter(dloca, [posa], dlsa)
                plsc.store_scatter(srca, [posb], s16b)
                plsc.store_scatter(dloca, [posb], dlsb)
                return jnp.minimum(offv1 + pcb, CAP - 1024)

            return lax.fori_loop(0, CE // (2 * L), grp, offv)

        NCH = E // CE
        issue(0, sbuf0, dbuf0, sem0)

        def half(h, offv):
            i1 = 2 * h + 1
            issue(i1, sbuf1, dbuf1, sem1)
            waitch(sbuf0, dbuf0, sem0)
            offv = scan_chunk(sbuf0, dbuf0, offv)

            @pl.when(i1 + 1 < NCH)
            def _():
                issue(i1 + 1, sbuf0, dbuf0, sem0)

            waitch(sbuf1, dbuf1, sem1)
            offv = scan_chunk(sbuf1, dbuf1, offv)
            return offv

        offv = lax.fori_loop(0, NCH // 2, half, jnp.zeros((L,), _i32))
        off = jnp.max(offv)

        def slgrp(j, off):
            idxv = off + iota
            plsc.store_scatter(srca, [idxv], base + j * L + iota)
            plsc.store_scatter(dloca, [idxv], j * L + iota)
            cur = plsc.load_gather(dega, [j * L + iota])
            plsc.store_scatter(dega, [j * L + iota], cur + 1.0)
            return off + L

        off = lax.fori_loop(0, nvalid // L, slgrp, off)

        target = ((off + K - 1) // K) * K
        for i in range(K // L):
            idxv = off + i * L + iota
            idxv = jnp.where(idxv < target, idxv, CAP - L + iota)
            plsc.store_scatter(srca, [idxv], jnp.zeros((L,), _i32))
            plsc.store_scatter(dloca, [idxv], jnp.full((L,), DUMMY, _i32))

        cntv[...] = lax.broadcast(target // K, (L,))
        pltpu.sync_copy(cntv, cnt_hbm.at[w])
        pltpu.sync_copy(dega.at[pl.ds(0, R)], deg_hbm.at[pl.ds(base, R)])
        pltpu.sync_copy(srca, srcl_hbm.at[w])
        pltpu.sync_copy(dloca, dlocl_hbm.at[w])

    return p1(src, dst)


@functools.lru_cache(maxsize=None)
def _make_agg(W):
    @functools.partial(
        pl.kernel,
        mesh=_mesh(),
        compiler_params=pltpu.CompilerParams(needs_layout_passes=False),
        out_type=jax.ShapeDtypeStruct((NPAD, W), _f32),
        scratch_types=[
            pltpu.VMEM((CAP,), _i32),
            pltpu.VMEM((CAP,), _i32),
            pltpu.VMEM((L,), _i32),
            pltpu.VMEM((ACCR, W), _f32),
            pltpu.VMEM((K, W), _f32),
            pltpu.VMEM((K, W), _f32),
            pltpu.VMEM((K, W), _f32),
            pltpu.SemaphoreType.DMA,
            pltpu.SemaphoreType.DMA,
            pltpu.SemaphoreType.DMA,
        ],
    )
    def agg(x_hbm, srcl_hbm, dlocl_hbm, cnt_hbm, s_hbm,
            srca, dloca, cntv, acc, rows0, rows1, rows2, sem0, sem1, sem2):
        iota = lax.iota(_i32, L)
        w = _wid()
        base = w * R
        pltpu.sync_copy(cnt_hbm.at[w], cntv)
        nc = jnp.max(cntv[...])
        pltpu.sync_copy(srcl_hbm.at[w], srca)
        pltpu.sync_copy(dlocl_hbm.at[w], dloca)

        def zrow(r, _):
            for j in range(W // L):
                acc[r, pl.ds(j * L, L)] = jnp.zeros((L,), _f32)
            return 0

        lax.fori_loop(0, ACCR, zrow, 0)

        bufs = ((rows0, sem0), (rows1, sem1), (rows2, sem2))

        def issue(i, b):
            pltpu.make_async_copy(
                x_hbm.at[srca.at[pl.ds(i * K, K)]], bufs[b][0], bufs[b][1]).start()

        def wait(b):
            pltpu.make_async_copy(
                x_hbm.at[srca.at[pl.ds(0, K)]], bufs[b][0], bufs[b][1]).wait()

        def process(i, rows):
            def grp(g, _):
                dl16 = dloca[pl.ds(i * K + g * L, L)]
                for j in range(0, L, 2):
                    rsp0 = dl16.at[lax.broadcast(j, (L,))].get(
                        mode="promise_in_bounds")
                    rsp1 = dl16.at[lax.broadcast(j + 1, (L,))].get(
                        mode="promise_in_bounds")
                    e0 = g * L + j
                    e1 = e0 + 1
                    xs0 = [rows[e0, pl.ds(c * L, L)] for c in range(W // L)]
                    xs1 = [rows[e1, pl.ds(c * L, L)] for c in range(W // L)]
                    for c in range(W // L):
                        plsc.addupdate_scatter(acc, [rsp0, c * L + iota], xs0[c])
                    for c in range(W // L):
                        plsc.addupdate_scatter(acc, [rsp1, c * L + iota], xs1[c])
                return 0

            lax.fori_loop(0, K // L, grp, 0)

        issue(0, 0)

        @pl.when(1 < nc)
        def _():
            issue(1, 1)

        def trip(h, _):
            for b in range(3):
                i = 3 * h + b

                @pl.when(i < nc)
                def _():
                    wait(b)

                    @pl.when(i + 2 < nc)
                    def _():
                        issue(i + 2, (b + 2) % 3)

                    process(i, bufs[b][0])
            return 0

        lax.fori_loop(0, (nc + 2) // 3, trip, 0)
        pltpu.sync_copy(acc.at[pl.ds(0, R)], s_hbm.at[pl.ds(base, R)])

    return agg


K2 = 64


def _dual_agg_call(pk, srcl, dlocl, cnt):
    W = 128

    @functools.partial(
        pl.kernel,
        mesh=_mesh(),
        compiler_params=pltpu.CompilerParams(needs_layout_passes=False),
        out_type=(jax.ShapeDtypeStruct((NPAD, W), _f32),
                  jax.ShapeDtypeStruct((NPAD, W), _f32)),
        scratch_types=[
            pltpu.VMEM((CAP,), _i32),
            pltpu.VMEM((K2,), _i32),
            pltpu.VMEM((K2,), _i32),
            pltpu.VMEM((L,), _i32),
            pltpu.VMEM((ACCR, W), _f32),
            pltpu.VMEM((ACCR, W), _f32),
            pltpu.VMEM((K2, W), _i32),
            pltpu.VMEM((K2, W), _i32),
            pltpu.SemaphoreType.DMA,
            pltpu.SemaphoreType.DMA,
        ])
    def agg2(pk_hbm, srcl_hbm, dlocl_hbm, cnt_hbm, sf_hbm, sc_hbm,
             srca, dv0, dv1, cntv, accF, accC, rows0, rows1, sem0, sem1):
        iota = lax.iota(_i32, L)
        w = _wid()
        base = w * R
        pltpu.sync_copy(cnt_hbm.at[w], cntv)
        nc2 = jnp.max(cntv[...]) * 2
        pltpu.sync_copy(srcl_hbm.at[w], srca)

        def zrow(r, _):
            for j in range(W // L):
                accF[r, pl.ds(j * L, L)] = jnp.zeros((L,), _f32)
                accC[r, pl.ds(j * L, L)] = jnp.zeros((L,), _f32)
            return 0
        lax.fori_loop(0, ACCR, zrow, 0)

        def issue(i, rows, dv, sem):
            pltpu.make_async_copy(
                pk_hbm.at[srca.at[pl.ds(i * K2, K2)]], rows, sem).start()
            pltpu.make_async_copy(
                dlocl_hbm.at[w, pl.ds(i * K2, K2)], dv, sem).start()

        def wait(rows, dv, sem):
            pltpu.make_async_copy(
                pk_hbm.at[srca.at[pl.ds(0, K2)]], rows, sem).wait()
            pltpu.make_async_copy(
                dlocl_hbm.at[w, pl.ds(0, K2)], dv, sem).wait()

        def process(rows, dv):
            def grp(g, _):
                dl16 = dv[pl.ds(g * L, L)]
                for j in range(0, L, 2):
                    r0 = dl16.at[lax.broadcast(j, (L,))].get(
                        mode="promise_in_bounds")
                    r1 = dl16.at[lax.broadcast(j + 1, (L,))].get(
                        mode="promise_in_bounds")
                    e0 = g * L + j
                    e1 = e0 + 1

                    def halves(e):
                        out = []
                        for c in range(W // L):
                            v = rows[e, pl.ds(c * L, L)]
                            vb = plsc.bitcast(v, jnp.bfloat16)
                            out.append(plsc.unpack(
                                vb, format=plsc.PackFormat.INTERLEAVED))
                        return out

                    h0 = halves(e0)
                    h1 = halves(e1)
                    for c in range(W // L):
                        plsc.addupdate_scatter(
                            accF, [r0, c * L + iota], h0[c][0])
                        plsc.addupdate_scatter(
                            accC, [r0, c * L + iota], h0[c][1])
                    for c in range(W // L):
                        plsc.addupdate_scatter(
                            accF, [r1, c * L + iota], h1[c][0])
                        plsc.addupdate_scatter(
                            accC, [r1, c * L + iota], h1[c][1])
                return 0
            lax.fori_loop(0, K2 // L, grp, 0)

        issue(0, rows0, dv0, sem0)

        def half(h, _):
            i0, i1 = 2 * h, 2 * h + 1

            @pl.when(i1 < nc2)
            def _():
                issue(i1, rows1, dv1, sem1)
            wait(rows0, dv0, sem0)
            process(rows0, dv0)

            @pl.when(i1 < nc2)
            def _():
                @pl.when(i1 + 1 < nc2)
                def _():
                    issue(i1 + 1, rows0, dv0, sem0)
                wait(rows1, dv1, sem1)
                process(rows1, dv1)
            return 0

        lax.fori_loop(0, (nc2 + 1) // 2, half, 0)
        pltpu.sync_copy(accF.at[pl.ds(0, R)], sf_hbm.at[pl.ds(base, R)])
        pltpu.sync_copy(accC.at[pl.ds(0, R)], sc_hbm.at[pl.ds(base, R)])

    return agg2(pk, srcl, dlocl, cnt)


def _rows(i, _=None):
    return (i, 0)


def _bcast(i, _=None):
    return (0, 0)


BLK = 512


def _tc_call(body, ins, blockable, out_widths):
    in_specs = [
        pl.BlockSpec((BLK, a.shape[1]), _rows) if b
        else pl.BlockSpec(a.shape, _bcast)
        for a, b in zip(ins, blockable)
    ]
    out_widths = [w if isinstance(w, tuple) else (w, _f32) for w in out_widths]
    out_shape = tuple(
        jax.ShapeDtypeStruct((NPAD, wd), dt) for wd, dt in out_widths)
    out_specs = tuple(pl.BlockSpec((BLK, wd), _rows) for wd, _ in out_widths)
    outs = pl.pallas_call(
        body,
        grid=(NPAD // BLK,),
        in_specs=in_specs,
        out_specs=out_specs,
        out_shape=out_shape,
    )(*ins)
    return outs


def _mm(a, b):
    return jnp.dot(a, b, preferred_element_type=_f32)


def kernel(feature, condition, edge_index,
           enc_f2h_W, enc_f2h_b, enc_c2h_W, enc_c2h_b, enc_h2h_W, enc_h2h_b,
           enc_mean_W, enc_mean_b, enc_logvar_W, enc_logvar_b,
           dec_z2h_W, dec_z2h_b, dec_c2h_W, dec_c2h_b, dec_h2h_W, dec_h2h_b,
           dec_out_W, dec_out_b):
    pad = NPAD - N
    fpad = jnp.pad(feature, ((0, pad), (0, 0)))
    cpad = jnp.pad(condition, ((0, pad), (0, 0)))
    noise = jax.random.normal(jax.random.key(1), (N, 64), _f32)
    npad_ = jnp.pad(noise, ((0, pad), (0, 0)))

    Whh1, Whh2 = enc_h2h_W[:128], enc_h2h_W[128:]
    Wdhh1, Wdhh2 = dec_h2h_W[:128], dec_h2h_W[128:]
    Wmlv = jnp.concatenate([enc_mean_W, enc_logvar_W], axis=1)
    bmlv = jnp.concatenate([enc_mean_b, enc_logvar_b]).reshape(1, 128)
    bf = enc_f2h_b.reshape(1, -1)
    bc = enc_c2h_b.reshape(1, -1)
    bh = enc_h2h_b.reshape(1, -1)
    bz = dec_z2h_b.reshape(1, -1)
    bdc = dec_c2h_b.reshape(1, -1)
    bdh = dec_h2h_b.reshape(1, -1)
    bout = dec_out_b.reshape(1, -1)

    deg, srcl, dlocl, cnt = _partition_call(edge_index[0], edge_index[1])
    degc = deg.reshape(NPAD, 1)

    agg128 = _make_agg(128)

    def tc0(deg_r, f_r, c_r, dinv_o, pk_o):
        dv = lax.rsqrt(jnp.maximum(deg_r[...], 1.0))
        dinv_o[...] = dv
        fb = lax.bitcast_convert_type(
            (f_r[...] * dv).astype(jnp.bfloat16), jnp.uint16).astype(jnp.uint32)
        cb = lax.bitcast_convert_type(
            (c_r[...] * dv).astype(jnp.bfloat16), jnp.uint16).astype(jnp.uint32)
        pk_o[...] = lax.bitcast_convert_type(fb | (cb << 16), jnp.int32)

    dinv, pk = _tc_call(tc0, [degc, fpad, cpad], [True] * 3,
                        [1, (128, jnp.int32)])

    s_f, s_c = _dual_agg_call(pk, srcl, dlocl, cnt)

    def tc1(sf_r, sc_r, dv_r, wf, bf_r, wc, bc_r, wdc, bdc_r, whh1, whh2,
            wdhh2, ts_o, t2a_o):
        dv = dv_r[...]
        f2h = jnp.tanh(_mm(dv * sf_r[...], wf[...]) + bf_r[...])
        cpre = dv * sc_r[...]
        c2h = jnp.tanh(_mm(cpre, wc[...]) + bc_r[...])
        dc2h = jnp.tanh(_mm(cpre, wdc[...]) + bdc_r[...])
        ts_o[...] = dv * (_mm(f2h, whh1[...]) + _mm(c2h, whh2[...]))
        t2a_o[...] = _mm(dc2h, wdhh2[...])

    ts, t2a = _tc_call(
        tc1,
        [s_f, s_c, dinv, enc_f2h_W, bf, enc_c2h_W, bc, dec_c2h_W, bdc,
         Whh1, Whh2, Wdhh2],
        [True, True, True] + [False] * 9,
        [128, 128])

    s_t = agg128(ts, srcl, dlocl, cnt)

    def tc2(st_r, dv_r, bh_r, wmlv, ms_o):
        dv = dv_r[...]
        h = jnp.tanh(dv * st_r[...] + bh_r[...])
        ms_o[...] = dv * _mm(h, wmlv[...])

    (ms,) = _tc_call(tc2, [s_t, dinv, bh, Wmlv],
                     [True, True, False, False], [128])

    s_m = agg128(ms, srcl, dlocl, cnt)

    def tc3(sm_r, dv_r, bmlv_r, nz_r, mean_o, logvar_o, z_o, zs_o):
        dv = dv_r[...]
        mlv = dv * sm_r[...] + bmlv_r[...]
        mean = mlv[:, :64]
        logvar = mlv[:, 64:]
        z = nz_r[...] * jnp.exp(0.5 * logvar) + mean
        mean_o[...] = mean
        logvar_o[...] = logvar
        z_o[...] = z
        zs_o[...] = jnp.concatenate(
            [dv * z, jnp.zeros_like(z)], axis=1)

    mean, logvar, z, zs128 = _tc_call(
        tc3, [s_m, dinv, bmlv, npad_],
        [True, True, False, True], [64, 64, 64, 128])

    s_z = agg128(zs128, srcl, dlocl, cnt)

    def tc4(sz_r, dv_r, wz, bz_r, wdhh1, t2a_r, t2s_o):
        dv = dv_r[...]
        z2h = jnp.tanh(_mm(dv * sz_r[..., :64], wz[...]) + bz_r[...])
        t2s_o[...] = dv * (_mm(z2h, wdhh1[...]) + t2a_r[...])

    (t2s,) = _tc_call(tc4, [s_z, dinv, dec_z2h_W, bz, Wdhh1, t2a],
                      [True, True, False, False, False, True], [128])

    s_t2 = agg128(t2s, srcl, dlocl, cnt)

    def tc5(st2_r, dv_r, bdh_r, wout, t3s_o):
        dv = dv_r[...]
        dh = jnp.tanh(dv * st2_r[...] + bdh_r[...])
        t3s_o[...] = dv * _mm(dh, wout[...])

    (t3s,) = _tc_call(tc5, [s_t2, dinv, bdh, dec_out_W],
                      [True, True, False, False], [128])

    s_o = agg128(t3s, srcl, dlocl, cnt)

    def tc6(so_r, dv_r, bout_r, out_o):
        out_o[...] = dv_r[...] * so_r[...] + bout_r[...]

    (outp,) = _tc_call(tc6, [s_o, dinv, bout], [True, True, False], [128])

    return (z[:N], mean[:N], logvar[:N], outp[:N])


# speedup vs baseline: 1.2279x; 1.0433x over previous
import functools

import jax
import jax.numpy as jnp
from jax import lax
from jax.experimental import pallas as pl
from jax.experimental.pallas import tpu as pltpu
from jax.experimental.pallas import tpu_sc as plsc

N = 10000
E = 320000
NC, NS, L = 2, 16, 16
NW = NC * NS
R = 320
NPAD = NW * R
CAP = 16384
K = 128
ACCR = 336
DUMMY = 320
CE = 4000

_mesh = lambda: plsc.VectorSubcoreMesh(core_axis_name="c", subcore_axis_name="s")

_f32 = jnp.float32
_i32 = jnp.int32


def _wid():
    return lax.axis_index("s") * NC + lax.axis_index("c")


def _partition_call(src, dst):
    @functools.partial(
        pl.kernel,
        mesh=_mesh(),
        compiler_params=pltpu.CompilerParams(needs_layout_passes=False),
        out_type=(
            jax.ShapeDtypeStruct((NPAD,), _f32),
            jax.ShapeDtypeStruct((NW, CAP), _i32),
            jax.ShapeDtypeStruct((NW, CAP), _i32),
            jax.ShapeDtypeStruct((NW, L), _i32),
        ),
        scratch_types=[
            pltpu.VMEM((CE,), _i32),
            pltpu.VMEM((CE,), _i32),
            pltpu.VMEM((CE,), _i32),
            pltpu.VMEM((CE,), _i32),
            pltpu.VMEM((ACCR,), _f32),
            pltpu.VMEM((CAP,), _i32),
            pltpu.VMEM((CAP,), _i32),
            pltpu.VMEM((L,), _i32),
            pltpu.SemaphoreType.DMA,
            pltpu.SemaphoreType.DMA,
        ],
    )
    def p1(src_hbm, dst_hbm, deg_hbm, srcl_hbm, dlocl_hbm, cnt_hbm,
           sbuf0, dbuf0, sbuf1, dbuf1, dega, srca, dloca, cntv, sem0, sem1):
        iota = lax.iota(_i32, L)
        w = _wid()
        base = w * R
        nvalid = jnp.minimum(R, N - base)

        for i in range(ACCR // L):
            dega[pl.ds(i * L, L)] = jnp.zeros((L,), _f32)

        def issue(ci, sb, db, sem):
            pltpu.make_async_copy(src_hbm.at[pl.ds(ci * CE, CE)], sb, sem).start()
            pltpu.make_async_copy(dst_hbm.at[pl.ds(ci * CE, CE)], db, sem).start()

        def waitch(sb, db, sem):
            pltpu.make_async_copy(src_hbm.at[pl.ds(0, CE)], sb, sem).wait()
            pltpu.make_async_copy(dst_hbm.at[pl.ds(0, CE)], db, sem).wait()

        def scan_chunk(sb, db, offv):
            def grp(gi2, offv):
                g0 = gi2 * 2
                s16a = sb[pl.ds(g0 * L, L)]
                d16a = db[pl.ds(g0 * L, L)]
                s16b = sb[pl.ds((g0 + 1) * L, L)]
                d16b = db[pl.ds((g0 + 1) * L, L)]
                dla = d16a - base
                dlb = d16b - base
                ma = (dla >= 0) & (dla < nvalid)
                mb = (dlb >= 0) & (dlb < nvalid)
                dlsa = jnp.where(ma, dla, DUMMY)
                dlsb = jnp.where(mb, dlb, DUMMY)
                cma = plsc.cumsum(ma.astype(_i32))
                cmb = plsc.cumsum(mb.astype(_i32))
                pca = plsc.all_reduce_population_count(ma)
                pcb = plsc.all_reduce_population_count(mb)
                posa = jnp.where(ma, offv + cma - 1, CAP - L + iota)
                offv1 = offv + pca
                posb = jnp.where(mb, offv1 + cmb - 1, CAP - L + iota)
                plsc.addupdate_scatter(dega, [dlsa], jnp.where(ma, 1.0, 0.0))
                plsc.addupdate_scatter(dega, [dlsb], jnp.where(mb, 1.0, 0.0))
                plsc.store_scatter(srca, [posa], s16a)
                plsc.store_scatter(dloca, [posa], dlsa)
                plsc.store_scatter(srca, [posb], s16b)
                plsc.store_scatter(dloca, [posb], dlsb)
                return jnp.minimum(offv1 + pcb, CAP - 1024)

            return lax.fori_loop(0, CE // (2 * L), grp, offv)

        NCH = E // CE
        issue(0, sbuf0, dbuf0, sem0)

        def half(h, offv):
            i1 = 2 * h + 1
            issue(i1, sbuf1, dbuf1, sem1)
            waitch(sbuf0, dbuf0, sem0)
            offv = scan_chunk(sbuf0, dbuf0, offv)

            @pl.when(i1 + 1 < NCH)
            def _():
                issue(i1 + 1, sbuf0, dbuf0, sem0)

            waitch(sbuf1, dbuf1, sem1)
            offv = scan_chunk(sbuf1, dbuf1, offv)
            return offv

        offv = lax.fori_loop(0, NCH // 2, half, jnp.zeros((L,), _i32))
        off = jnp.max(offv)

        def slgrp(j, off):
            idxv = off + iota
            plsc.store_scatter(srca, [idxv], base + j * L + iota)
            plsc.store_scatter(dloca, [idxv], j * L + iota)
            cur = plsc.load_gather(dega, [j * L + iota])
            plsc.store_scatter(dega, [j * L + iota], cur + 1.0)
            return off + L

        off = lax.fori_loop(0, nvalid // L, slgrp, off)

        target = ((off + K - 1) // K) * K
        for i in range(K // L):
            idxv = off + i * L + iota
            idxv = jnp.where(idxv < target, idxv, CAP - L + iota)
            plsc.store_scatter(srca, [idxv], jnp.zeros((L,), _i32))
            plsc.store_scatter(dloca, [idxv], jnp.full((L,), DUMMY, _i32))

        cntv[...] = lax.broadcast(target // K, (L,))
        pltpu.sync_copy(cntv, cnt_hbm.at[w])
        pltpu.sync_copy(dega.at[pl.ds(0, R)], deg_hbm.at[pl.ds(base, R)])
        pltpu.sync_copy(srca, srcl_hbm.at[w])
        pltpu.sync_copy(dloca, dlocl_hbm.at[w])

    return p1(src, dst)


@functools.lru_cache(maxsize=None)
def _make_agg(W, tc_tiling=True):
    @functools.partial(
        pl.kernel,
        mesh=_mesh(),
        compiler_params=pltpu.CompilerParams(
            needs_layout_passes=False, use_tc_tiling_on_sc=tc_tiling),
        out_type=jax.ShapeDtypeStruct((NPAD, W), _f32),
        scratch_types=[
            pltpu.VMEM((CAP,), _i32),
            pltpu.VMEM((CAP,), _i32),
            pltpu.VMEM((L,), _i32),
            pltpu.VMEM((ACCR, W), _f32),
            pltpu.VMEM((K, W), _f32),
            pltpu.VMEM((K, W), _f32),
            pltpu.VMEM((K, W), _f32),
            pltpu.SemaphoreType.DMA,
            pltpu.SemaphoreType.DMA,
            pltpu.SemaphoreType.DMA,
        ],
    )
    def agg(x_hbm, srcl_hbm, dlocl_hbm, cnt_hbm, s_hbm,
            srca, dloca, cntv, acc, rows0, rows1, rows2, sem0, sem1, sem2):
        iota = lax.iota(_i32, L)
        w = _wid()
        base = w * R
        pltpu.sync_copy(cnt_hbm.at[w], cntv)
        nc = jnp.max(cntv[...])
        pltpu.sync_copy(srcl_hbm.at[w], srca)
        pltpu.sync_copy(dlocl_hbm.at[w], dloca)

        def zrow(r, _):
            for j in range(W // L):
                acc[r, pl.ds(j * L, L)] = jnp.zeros((L,), _f32)
            return 0

        lax.fori_loop(0, ACCR, zrow, 0)

        bufs = ((rows0, sem0), (rows1, sem1), (rows2, sem2))

        def issue(i, b):
            pltpu.make_async_copy(
                x_hbm.at[srca.at[pl.ds(i * K, K)]], bufs[b][0], bufs[b][1]).start()

        def wait(b):
            pltpu.make_async_copy(
                x_hbm.at[srca.at[pl.ds(0, K)]], bufs[b][0], bufs[b][1]).wait()

        def process(i, rows):
            def grp(g, _):
                dl16 = dloca[pl.ds(i * K + g * L, L)]
                for j in range(0, L, 2):
                    rsp0 = dl16.at[lax.broadcast(j, (L,))].get(
                        mode="promise_in_bounds")
                    rsp1 = dl16.at[lax.broadcast(j + 1, (L,))].get(
                        mode="promise_in_bounds")
                    e0 = g * L + j
                    e1 = e0 + 1
                    xs0 = [rows[e0, pl.ds(c * L, L)] for c in range(W // L)]
                    xs1 = [rows[e1, pl.ds(c * L, L)] for c in range(W // L)]
                    for c in range(W // L):
                        plsc.addupdate_scatter(acc, [rsp0, c * L + iota], xs0[c])
                    for c in range(W // L):
                        plsc.addupdate_scatter(acc, [rsp1, c * L + iota], xs1[c])
                return 0

            lax.fori_loop(0, K // L, grp, 0)

        issue(0, 0)

        @pl.when(1 < nc)
        def _():
            issue(1, 1)

        def trip(h, _):
            for b in range(3):
                i = 3 * h + b

                @pl.when(i < nc)
                def _():
                    wait(b)

                    @pl.when(i + 2 < nc)
                    def _():
                        issue(i + 2, (b + 2) % 3)

                    process(i, bufs[b][0])
            return 0

        lax.fori_loop(0, (nc + 2) // 3, trip, 0)
        pltpu.sync_copy(acc.at[pl.ds(0, R)], s_hbm.at[pl.ds(base, R)])

    return agg


K2 = 64


def _dual_agg_call(pk, srcl, dlocl, cnt):
    W = 128

    @functools.partial(
        pl.kernel,
        mesh=_mesh(),
        compiler_params=pltpu.CompilerParams(needs_layout_passes=False),
        out_type=(jax.ShapeDtypeStruct((NPAD, W), _f32),
                  jax.ShapeDtypeStruct((NPAD, W), _f32)),
        scratch_types=[
            pltpu.VMEM((CAP,), _i32),
            pltpu.VMEM((K2,), _i32),
            pltpu.VMEM((K2,), _i32),
            pltpu.VMEM((L,), _i32),
            pltpu.VMEM((ACCR, W), _f32),
            pltpu.VMEM((ACCR, W), _f32),
            pltpu.VMEM((K2, W), _i32),
            pltpu.VMEM((K2, W), _i32),
            pltpu.SemaphoreType.DMA,
            pltpu.SemaphoreType.DMA,
        ])
    def agg2(pk_hbm, srcl_hbm, dlocl_hbm, cnt_hbm, sf_hbm, sc_hbm,
             srca, dv0, dv1, cntv, accF, accC, rows0, rows1, sem0, sem1):
        iota = lax.iota(_i32, L)
        w = _wid()
        base = w * R
        pltpu.sync_copy(cnt_hbm.at[w], cntv)
        nc2 = jnp.max(cntv[...]) * 2
        pltpu.sync_copy(srcl_hbm.at[w], srca)

        def zrow(r, _):
            for j in range(W // L):
                accF[r, pl.ds(j * L, L)] = jnp.zeros((L,), _f32)
                accC[r, pl.ds(j * L, L)] = jnp.zeros((L,), _f32)
            return 0
        lax.fori_loop(0, ACCR, zrow, 0)

        def issue(i, rows, dv, sem):
            pltpu.make_async_copy(
                pk_hbm.at[srca.at[pl.ds(i * K2, K2)]], rows, sem).start()
            pltpu.make_async_copy(
                dlocl_hbm.at[w, pl.ds(i * K2, K2)], dv, sem).start()

        def wait(rows, dv, sem):
            pltpu.make_async_copy(
                pk_hbm.at[srca.at[pl.ds(0, K2)]], rows, sem).wait()
            pltpu.make_async_copy(
                dlocl_hbm.at[w, pl.ds(0, K2)], dv, sem).wait()

        def process(rows, dv):
            def grp(g, _):
                dl16 = dv[pl.ds(g * L, L)]
                for j in range(0, L, 2):
                    r0 = dl16.at[lax.broadcast(j, (L,))].get(
                        mode="promise_in_bounds")
                    r1 = dl16.at[lax.broadcast(j + 1, (L,))].get(
                        mode="promise_in_bounds")
                    e0 = g * L + j
                    e1 = e0 + 1

                    def halves(e):
                        out = []
                        for c in range(W // L):
                            v = rows[e, pl.ds(c * L, L)]
                            vb = plsc.bitcast(v, jnp.bfloat16)
                            out.append(plsc.unpack(
                                vb, format=plsc.PackFormat.INTERLEAVED))
                        return out

                    h0 = halves(e0)
                    h1 = halves(e1)
                    for c in range(W // L):
                        plsc.addupdate_scatter(
                            accF, [r0, c * L + iota], h0[c][0])
                        plsc.addupdate_scatter(
                            accC, [r0, c * L + iota], h0[c][1])
                    for c in range(W // L):
                        plsc.addupdate_scatter(
                            accF, [r1, c * L + iota], h1[c][0])
                        plsc.addupdate_scatter(
                            accC, [r1, c * L + iota], h1[c][1])
                return 0
            lax.fori_loop(0, K2 // L, grp, 0)

        issue(0, rows0, dv0, sem0)

        def half(h, _):
            i0, i1 = 2 * h, 2 * h + 1

            @pl.when(i1 < nc2)
            def _():
                issue(i1, rows1, dv1, sem1)
            wait(rows0, dv0, sem0)
            process(rows0, dv0)

            @pl.when(i1 < nc2)
            def _():
                @pl.when(i1 + 1 < nc2)
                def _():
                    issue(i1 + 1, rows0, dv0, sem0)
                wait(rows1, dv1, sem1)
                process(rows1, dv1)
            return 0

        lax.fori_loop(0, (nc2 + 1) // 2, half, 0)
        pltpu.sync_copy(accF.at[pl.ds(0, R)], sf_hbm.at[pl.ds(base, R)])
        pltpu.sync_copy(accC.at[pl.ds(0, R)], sc_hbm.at[pl.ds(base, R)])

    return agg2(pk, srcl, dlocl, cnt)


def _rows(i, _=None):
    return (i, 0)


def _bcast(i, _=None):
    return (0, 0)


BLK = 512


def _tc_call(body, ins, blockable, out_widths):
    in_specs = [
        pl.BlockSpec((BLK, a.shape[1]), _rows) if b
        else pl.BlockSpec(a.shape, _bcast)
        for a, b in zip(ins, blockable)
    ]
    out_widths = [w if isinstance(w, tuple) else (w, _f32) for w in out_widths]
    out_shape = tuple(
        jax.ShapeDtypeStruct((NPAD, wd), dt) for wd, dt in out_widths)
    out_specs = tuple(pl.BlockSpec((BLK, wd), _rows) for wd, _ in out_widths)
    outs = pl.pallas_call(
        body,
        grid=(NPAD // BLK,),
        in_specs=in_specs,
        out_specs=out_specs,
        out_shape=out_shape,
    )(*ins)
    return outs


def _mm(a, b):
    return jnp.dot(a, b, preferred_element_type=_f32)


def kernel(feature, condition, edge_index,
           enc_f2h_W, enc_f2h_b, enc_c2h_W, enc_c2h_b, enc_h2h_W, enc_h2h_b,
           enc_mean_W, enc_mean_b, enc_logvar_W, enc_logvar_b,
           dec_z2h_W, dec_z2h_b, dec_c2h_W, dec_c2h_b, dec_h2h_W, dec_h2h_b,
           dec_out_W, dec_out_b):
    pad = NPAD - N
    fpad = jnp.pad(feature, ((0, pad), (0, 0)))
    cpad = jnp.pad(condition, ((0, pad), (0, 0)))
    noise = jax.random.normal(jax.random.key(1), (N, 64), _f32)
    npad_ = jnp.pad(noise, ((0, pad), (0, 0)))

    Whh1, Whh2 = enc_h2h_W[:128], enc_h2h_W[128:]
    Wdhh1, Wdhh2 = dec_h2h_W[:128], dec_h2h_W[128:]
    Wmlv = jnp.concatenate([enc_mean_W, enc_logvar_W], axis=1)
    bmlv = jnp.concatenate([enc_mean_b, enc_logvar_b]).reshape(1, 128)
    bf = enc_f2h_b.reshape(1, -1)
    bc = enc_c2h_b.reshape(1, -1)
    bh = enc_h2h_b.reshape(1, -1)
    bz = dec_z2h_b.reshape(1, -1)
    bdc = dec_c2h_b.reshape(1, -1)
    bdh = dec_h2h_b.reshape(1, -1)
    bout = dec_out_b.reshape(1, -1)

    deg, srcl, dlocl, cnt = _partition_call(edge_index[0], edge_index[1])
    degc = deg.reshape(NPAD, 1)

    agg128 = _make_agg(128)

    def tc0(deg_r, f_r, c_r, dinv_o, pk_o):
        dv = lax.rsqrt(jnp.maximum(deg_r[...], 1.0))
        dinv_o[...] = dv
        fb = lax.bitcast_convert_type(
            (f_r[...] * dv).astype(jnp.bfloat16), jnp.uint16).astype(jnp.uint32)
        cb = lax.bitcast_convert_type(
            (c_r[...] * dv).astype(jnp.bfloat16), jnp.uint16).astype(jnp.uint32)
        pk_o[...] = lax.bitcast_convert_type(fb | (cb << 16), jnp.int32)

    dinv, pk = _tc_call(tc0, [degc, fpad, cpad], [True] * 3,
                        [1, (128, jnp.int32)])

    s_f, s_c = _dual_agg_call(pk, srcl, dlocl, cnt)

    def tc1(sf_r, sc_r, dv_r, wf, bf_r, wc, bc_r, wdc, bdc_r, whh1, whh2,
            wdhh2, ts_o, t2a_o):
        dv = dv_r[...]
        f2h = jnp.tanh(_mm(dv * sf_r[...], wf[...]) + bf_r[...])
        cpre = dv * sc_r[...]
        c2h = jnp.tanh(_mm(cpre, wc[...]) + bc_r[...])
        dc2h = jnp.tanh(_mm(cpre, wdc[...]) + bdc_r[...])
        ts_o[...] = dv * (_mm(f2h, whh1[...]) + _mm(c2h, whh2[...]))
        t2a_o[...] = _mm(dc2h, wdhh2[...])

    ts, t2a = _tc_call(
        tc1,
        [s_f, s_c, dinv, enc_f2h_W, bf, enc_c2h_W, bc, dec_c2h_W, bdc,
         Whh1, Whh2, Wdhh2],
        [True, True, True] + [False] * 9,
        [128, 128])

    s_t = agg128(ts, srcl, dlocl, cnt)

    def tc2(st_r, dv_r, bh_r, wmlv, ms_o):
        dv = dv_r[...]
        h = jnp.tanh(dv * st_r[...] + bh_r[...])
        ms_o[...] = dv * _mm(h, wmlv[...])

    (ms,) = _tc_call(tc2, [s_t, dinv, bh, Wmlv],
                     [True, True, False, False], [128])

    s_m = agg128(ms, srcl, dlocl, cnt)

    def tc3(sm_r, dv_r, bmlv_r, nz_r, mean_o, logvar_o, z_o, zs_o):
        dv = dv_r[...]
        mlv = dv * sm_r[...] + bmlv_r[...]
        mean = mlv[:, :64]
        logvar = mlv[:, 64:]
        z = nz_r[...] * jnp.exp(0.5 * logvar) + mean
        mean_o[...] = mean
        logvar_o[...] = logvar
        z_o[...] = z
        zs_o[...] = dv * z

    mean, logvar, z, zs = _tc_call(
        tc3, [s_m, dinv, bmlv, npad_],
        [True, True, False, True], [64, 64, 64, 64])

    s_z = _make_agg(64, tc_tiling=False)(zs, srcl, dlocl, cnt)

    def tc4(sz_r, dv_r, wz, bz_r, wdhh1, t2a_r, t2s_o):
        dv = dv_r[...]
        z2h = jnp.tanh(_mm(dv * sz_r[...], wz[...]) + bz_r[...])
        t2s_o[...] = dv * (_mm(z2h, wdhh1[...]) + t2a_r[...])

    (t2s,) = _tc_call(tc4, [s_z, dinv, dec_z2h_W, bz, Wdhh1, t2a],
                      [True, True, False, False, False, True], [128])

    s_t2 = agg128(t2s, srcl, dlocl, cnt)

    def tc5(st2_r, dv_r, bdh_r, wout, t3s_o):
        dv = dv_r[...]
        dh = jnp.tanh(dv * st2_r[...] + bdh_r[...])
        t3s_o[...] = dv * _mm(dh, wout[...])

    (t3s,) = _tc_call(tc5, [s_t2, dinv, bdh, dec_out_W],
                      [True, True, False, False], [128])

    s_o = agg128(t3s, srcl, dlocl, cnt)

    def tc6(so_r, dv_r, bout_r, out_o):
        out_o[...] = dv_r[...] * so_r[...] + bout_r[...]

    (outp,) = _tc_call(tc6, [s_o, dinv, bout], [True, True, False], [128])

    return (z[:N], mean[:N], logvar[:N], outp[:N])


# speedup vs baseline: 1.4914x; 1.2146x over previous
import functools

import jax
import jax.numpy as jnp
from jax import lax
from jax.experimental import pallas as pl
from jax.experimental.pallas import tpu as pltpu
from jax.experimental.pallas import tpu_sc as plsc

N = 10000
E = 320000
NC, NS, L = 2, 16, 16
NW = NC * NS
R = 320
NPAD = NW * R
CAP = 16384
K = 128
ACCR = 336
DUMMY = 320
CE = 6400

_mesh = lambda: plsc.VectorSubcoreMesh(core_axis_name="c", subcore_axis_name="s")

_f32 = jnp.float32
_i32 = jnp.int32


def _wid():
    return lax.axis_index("s") * NC + lax.axis_index("c")


def _partition_call(src, dst):
    @functools.partial(
        pl.kernel,
        mesh=_mesh(),
        compiler_params=pltpu.CompilerParams(needs_layout_passes=False),
        out_type=(
            jax.ShapeDtypeStruct((NPAD,), _f32),
            jax.ShapeDtypeStruct((NW, CAP), _i32),
            jax.ShapeDtypeStruct((NW, CAP), _i32),
            jax.ShapeDtypeStruct((NW, L), _i32),
        ),
        scratch_types=[
            pltpu.VMEM((CE,), _i32),
            pltpu.VMEM((CE,), _i32),
            pltpu.VMEM((CE,), _i32),
            pltpu.VMEM((CE,), _i32),
            pltpu.VMEM((ACCR,), _f32),
            pltpu.VMEM((CAP,), _i32),
            pltpu.VMEM((CAP,), _i32),
            pltpu.VMEM((L,), _i32),
            pltpu.SemaphoreType.DMA,
            pltpu.SemaphoreType.DMA,
        ],
    )
    def p1(src_hbm, dst_hbm, deg_hbm, srcl_hbm, dlocl_hbm, cnt_hbm,
           sbuf0, dbuf0, sbuf1, dbuf1, dega, srca, dloca, cntv, sem0, sem1):
        iota = lax.iota(_i32, L)
        w = _wid()
        base = w * R
        nvalid = jnp.minimum(R, N - base)

        for i in range(ACCR // L):
            dega[pl.ds(i * L, L)] = jnp.zeros((L,), _f32)

        def issue(ci, sb, db, sem):
            pltpu.make_async_copy(src_hbm.at[pl.ds(ci * CE, CE)], sb, sem).start()
            pltpu.make_async_copy(dst_hbm.at[pl.ds(ci * CE, CE)], db, sem).start()

        def waitch(sb, db, sem):
            pltpu.make_async_copy(src_hbm.at[pl.ds(0, CE)], sb, sem).wait()
            pltpu.make_async_copy(dst_hbm.at[pl.ds(0, CE)], db, sem).wait()

        def scan_chunk(sb, db, offv):
            def grp(gi4, offv):
                g0 = gi4 * 4
                ss = [sb[pl.ds((g0 + k) * L, L)] for k in range(4)]
                dd = [db[pl.ds((g0 + k) * L, L)] for k in range(4)]
                dls_, ms_, cms_, pcs_ = [], [], [], []
                for k in range(4):
                    dl = dd[k] - base
                    m = (dl >= 0) & (dl < nvalid)
                    ms_.append(m)
                    dls_.append(jnp.where(m, dl, DUMMY))
                    cms_.append(plsc.cumsum(m.astype(_i32)))
                    pcs_.append(plsc.all_reduce_population_count(m))
                off = offv
                for k in range(4):
                    pos = jnp.where(ms_[k], off + cms_[k] - 1, CAP - L + iota)
                    plsc.addupdate_scatter(
                        dega, [dls_[k]], jnp.where(ms_[k], 1.0, 0.0))
                    plsc.store_scatter(srca, [pos], ss[k])
                    plsc.store_scatter(dloca, [pos], dls_[k])
                    off = off + pcs_[k]
                return jnp.minimum(off, CAP - 1024)

            return lax.fori_loop(0, CE // (4 * L), grp, offv)

        NCH = E // CE
        issue(0, sbuf0, dbuf0, sem0)

        def half(h, offv):
            i1 = 2 * h + 1
            issue(i1, sbuf1, dbuf1, sem1)
            waitch(sbuf0, dbuf0, sem0)
            offv = scan_chunk(sbuf0, dbuf0, offv)

            @pl.when(i1 + 1 < NCH)
            def _():
                issue(i1 + 1, sbuf0, dbuf0, sem0)

            waitch(sbuf1, dbuf1, sem1)
            offv = scan_chunk(sbuf1, dbuf1, offv)
            return offv

        offv = lax.fori_loop(0, NCH // 2, half, jnp.zeros((L,), _i32))
        off = jnp.max(offv)

        def slgrp(j, off):
            idxv = off + iota
            plsc.store_scatter(srca, [idxv], base + j * L + iota)
            plsc.store_scatter(dloca, [idxv], j * L + iota)
            cur = plsc.load_gather(dega, [j * L + iota])
            plsc.store_scatter(dega, [j * L + iota], cur + 1.0)
            return off + L

        off = lax.fori_loop(0, nvalid // L, slgrp, off)

        target = ((off + K - 1) // K) * K
        for i in range(K // L):
            idxv = off + i * L + iota
            idxv = jnp.where(idxv < target, idxv, CAP - L + iota)
            plsc.store_scatter(srca, [idxv], jnp.zeros((L,), _i32))
            plsc.store_scatter(dloca, [idxv], jnp.full((L,), DUMMY, _i32))

        cntv[...] = lax.broadcast(target // K, (L,))
        pltpu.sync_copy(cntv, cnt_hbm.at[w])
        pltpu.sync_copy(dega.at[pl.ds(0, R)], deg_hbm.at[pl.ds(base, R)])
        pltpu.sync_copy(srca, srcl_hbm.at[w])
        pltpu.sync_copy(dloca, dlocl_hbm.at[w])

    return p1(src, dst)


@functools.lru_cache(maxsize=None)
def _make_agg(W, tc_tiling=True):
    @functools.partial(
        pl.kernel,
        mesh=_mesh(),
        compiler_params=pltpu.CompilerParams(
            needs_layout_passes=False, use_tc_tiling_on_sc=tc_tiling),
        out_type=jax.ShapeDtypeStruct((NPAD, W), _f32),
        scratch_types=[
            pltpu.VMEM((CAP,), _i32),
            pltpu.VMEM((CAP,), _i32),
            pltpu.VMEM((L,), _i32),
            pltpu.VMEM((ACCR, W), _f32),
            pltpu.VMEM((K, W), _f32),
            pltpu.VMEM((K, W), _f32),
            pltpu.VMEM((K, W), _f32),
            pltpu.SemaphoreType.DMA,
            pltpu.SemaphoreType.DMA,
            pltpu.SemaphoreType.DMA,
        ],
    )
    def agg(x_hbm, srcl_hbm, dlocl_hbm, cnt_hbm, s_hbm,
            srca, dloca, cntv, acc, rows0, rows1, rows2, sem0, sem1, sem2):
        iota = lax.iota(_i32, L)
        w = _wid()
        base = w * R
        pltpu.sync_copy(cnt_hbm.at[w], cntv)
        nc = jnp.max(cntv[...])
        pltpu.sync_copy(srcl_hbm.at[w], srca)
        pltpu.sync_copy(dlocl_hbm.at[w], dloca)

        def zrow(r, _):
            for j in range(W // L):
                acc[r, pl.ds(j * L, L)] = jnp.zeros((L,), _f32)
            return 0

        lax.fori_loop(0, ACCR, zrow, 0)

        bufs = ((rows0, sem0), (rows1, sem1), (rows2, sem2))

        def issue(i, b):
            pltpu.make_async_copy(
                x_hbm.at[srca.at[pl.ds(i * K, K)]], bufs[b][0], bufs[b][1]).start()

        def wait(b):
            pltpu.make_async_copy(
                x_hbm.at[srca.at[pl.ds(0, K)]], bufs[b][0], bufs[b][1]).wait()

        def process(i, rows):
            def grp(g, _):
                dl16 = dloca[pl.ds(i * K + g * L, L)]
                for j in range(0, L, 2):
                    rsp0 = dl16.at[lax.broadcast(j, (L,))].get(
                        mode="promise_in_bounds")
                    rsp1 = dl16.at[lax.broadcast(j + 1, (L,))].get(
                        mode="promise_in_bounds")
                    e0 = g * L + j
                    e1 = e0 + 1
                    xs0 = [rows[e0, pl.ds(c * L, L)] for c in range(W // L)]
                    xs1 = [rows[e1, pl.ds(c * L, L)] for c in range(W // L)]
                    for c in range(W // L):
                        plsc.addupdate_scatter(acc, [rsp0, c * L + iota], xs0[c])
                    for c in range(W // L):
                        plsc.addupdate_scatter(acc, [rsp1, c * L + iota], xs1[c])
                return 0

            lax.fori_loop(0, K // L, grp, 0)

        issue(0, 0)

        @pl.when(1 < nc)
        def _():
            issue(1, 1)

        def trip(h, _):
            for b in range(3):
                i = 3 * h + b

                @pl.when(i < nc)
                def _():
                    wait(b)

                    @pl.when(i + 2 < nc)
                    def _():
                        issue(i + 2, (b + 2) % 3)

                    process(i, bufs[b][0])
            return 0

        lax.fori_loop(0, (nc + 2) // 3, trip, 0)
        pltpu.sync_copy(acc.at[pl.ds(0, R)], s_hbm.at[pl.ds(base, R)])

    return agg


@functools.lru_cache(maxsize=None)
def _make_agg_pk(W):
    HW = W // 2

    @functools.partial(
        pl.kernel,
        mesh=_mesh(),
        compiler_params=pltpu.CompilerParams(
            needs_layout_passes=False, use_tc_tiling_on_sc=False),
        out_type=jax.ShapeDtypeStruct((NPAD, W), _f32),
        scratch_types=[
            pltpu.VMEM((CAP,), _i32),
            pltpu.VMEM((CAP,), _i32),
            pltpu.VMEM((L,), _i32),
            pltpu.VMEM((ACCR, W), _f32),
            pltpu.VMEM((K, HW), _i32),
            pltpu.VMEM((K, HW), _i32),
            pltpu.VMEM((K, HW), _i32),
            pltpu.SemaphoreType.DMA,
            pltpu.SemaphoreType.DMA,
            pltpu.SemaphoreType.DMA,
        ],
    )
    def agg(x_hbm, srcl_hbm, dlocl_hbm, cnt_hbm, s_hbm,
            srca, dloca, cntv, acc, rows0, rows1, rows2, sem0, sem1, sem2):
        iota = lax.iota(_i32, L)
        w = _wid()
        base = w * R
        pltpu.sync_copy(cnt_hbm.at[w], cntv)
        nc = jnp.max(cntv[...])
        pltpu.sync_copy(srcl_hbm.at[w], srca)
        pltpu.sync_copy(dlocl_hbm.at[w], dloca)

        def zrow(r, _):
            for j in range(W // L):
                acc[r, pl.ds(j * L, L)] = jnp.zeros((L,), _f32)
            return 0

        lax.fori_loop(0, ACCR, zrow, 0)

        bufs = ((rows0, sem0), (rows1, sem1), (rows2, sem2))

        def issue(i, b):
            pltpu.make_async_copy(
                x_hbm.at[srca.at[pl.ds(i * K, K)]], bufs[b][0], bufs[b][1]).start()

        def wait(b):
            pltpu.make_async_copy(
                x_hbm.at[srca.at[pl.ds(0, K)]], bufs[b][0], bufs[b][1]).wait()

        def process(i, rows):
            def grp(g, _):
                dl16 = dloca[pl.ds(i * K + g * L, L)]
                for j in range(0, L, 2):
                    r0 = dl16.at[lax.broadcast(j, (L,))].get(
                        mode="promise_in_bounds")
                    r1 = dl16.at[lax.broadcast(j + 1, (L,))].get(
                        mode="promise_in_bounds")
                    e0 = g * L + j
                    e1 = e0 + 1

                    def halves(e):
                        out = []
                        for c in range(HW // L):
                            v = rows[e, pl.ds(c * L, L)]
                            vb = plsc.bitcast(v, jnp.bfloat16)
                            out.append(plsc.unpack(
                                vb, format=plsc.PackFormat.INTERLEAVED))
                        return out

                    h0 = halves(e0)
                    h1 = halves(e1)
                    for c in range(HW // L):
                        plsc.addupdate_scatter(
                            acc, [r0, c * L + iota], h0[c][0])
                        plsc.addupdate_scatter(
                            acc, [r0, HW + c * L + iota], h0[c][1])
                    for c in range(HW // L):
                        plsc.addupdate_scatter(
                            acc, [r1, c * L + iota], h1[c][0])
                        plsc.addupdate_scatter(
                            acc, [r1, HW + c * L + iota], h1[c][1])
                return 0

            lax.fori_loop(0, K // L, grp, 0)

        issue(0, 0)

        @pl.when(1 < nc)
        def _():
            issue(1, 1)

        def trip(h, _):
            for b in range(3):
                i = 3 * h + b

                @pl.when(i < nc)
                def _():
                    wait(b)

                    @pl.when(i + 2 < nc)
                    def _():
                        issue(i + 2, (b + 2) % 3)

                    process(i, bufs[b][0])
            return 0

        lax.fori_loop(0, (nc + 2) // 3, trip, 0)
        pltpu.sync_copy(acc.at[pl.ds(0, R)], s_hbm.at[pl.ds(base, R)])

    return agg


K2 = 64


def _dual_agg_call(pk, srcl, dlocl, cnt):
    W = 128

    @functools.partial(
        pl.kernel,
        mesh=_mesh(),
        compiler_params=pltpu.CompilerParams(needs_layout_passes=False),
        out_type=(jax.ShapeDtypeStruct((NPAD, W), _f32),
                  jax.ShapeDtypeStruct((NPAD, W), _f32)),
        scratch_types=[
            pltpu.VMEM((CAP,), _i32),
            pltpu.VMEM((K2,), _i32),
            pltpu.VMEM((K2,), _i32),
            pltpu.VMEM((L,), _i32),
            pltpu.VMEM((ACCR, W), _f32),
            pltpu.VMEM((ACCR, W), _f32),
            pltpu.VMEM((K2, W), _i32),
            pltpu.VMEM((K2, W), _i32),
            pltpu.SemaphoreType.DMA,
            pltpu.SemaphoreType.DMA,
        ])
    def agg2(pk_hbm, srcl_hbm, dlocl_hbm, cnt_hbm, sf_hbm, sc_hbm,
             srca, dv0, dv1, cntv, accF, accC, rows0, rows1, sem0, sem1):
        iota = lax.iota(_i32, L)
        w = _wid()
        base = w * R
        pltpu.sync_copy(cnt_hbm.at[w], cntv)
        nc2 = jnp.max(cntv[...]) * 2
        pltpu.sync_copy(srcl_hbm.at[w], srca)

        def zrow(r, _):
            for j in range(W // L):
                accF[r, pl.ds(j * L, L)] = jnp.zeros((L,), _f32)
                accC[r, pl.ds(j * L, L)] = jnp.zeros((L,), _f32)
            return 0
        lax.fori_loop(0, ACCR, zrow, 0)

        def issue(i, rows, dv, sem):
            pltpu.make_async_copy(
                pk_hbm.at[srca.at[pl.ds(i * K2, K2)]], rows, sem).start()
            pltpu.make_async_copy(
                dlocl_hbm.at[w, pl.ds(i * K2, K2)], dv, sem).start()

        def wait(rows, dv, sem):
            pltpu.make_async_copy(
                pk_hbm.at[srca.at[pl.ds(0, K2)]], rows, sem).wait()
            pltpu.make_async_copy(
                dlocl_hbm.at[w, pl.ds(0, K2)], dv, sem).wait()

        def process(rows, dv):
            def grp(g, _):
                dl16 = dv[pl.ds(g * L, L)]
                for j in range(0, L, 2):
                    r0 = dl16.at[lax.broadcast(j, (L,))].get(
                        mode="promise_in_bounds")
                    r1 = dl16.at[lax.broadcast(j + 1, (L,))].get(
                        mode="promise_in_bounds")
                    e0 = g * L + j
                    e1 = e0 + 1

                    def halves(e):
                        out = []
                        for c in range(W // L):
                            v = rows[e, pl.ds(c * L, L)]
                            vb = plsc.bitcast(v, jnp.bfloat16)
                            out.append(plsc.unpack(
                                vb, format=plsc.PackFormat.INTERLEAVED))
                        return out

                    h0 = halves(e0)
                    h1 = halves(e1)
                    for c in range(W // L):
                        plsc.addupdate_scatter(
                            accF, [r0, c * L + iota], h0[c][0])
                        plsc.addupdate_scatter(
                            accC, [r0, c * L + iota], h0[c][1])
                    for c in range(W // L):
                        plsc.addupdate_scatter(
                            accF, [r1, c * L + iota], h1[c][0])
                        plsc.addupdate_scatter(
                            accC, [r1, c * L + iota], h1[c][1])
                return 0
            lax.fori_loop(0, K2 // L, grp, 0)

        issue(0, rows0, dv0, sem0)

        def half(h, _):
            i0, i1 = 2 * h, 2 * h + 1

            @pl.when(i1 < nc2)
            def _():
                issue(i1, rows1, dv1, sem1)
            wait(rows0, dv0, sem0)
            process(rows0, dv0)

            @pl.when(i1 < nc2)
            def _():
                @pl.when(i1 + 1 < nc2)
                def _():
                    issue(i1 + 1, rows0, dv0, sem0)
                wait(rows1, dv1, sem1)
                process(rows1, dv1)
            return 0

        lax.fori_loop(0, (nc2 + 1) // 2, half, 0)
        pltpu.sync_copy(accF.at[pl.ds(0, R)], sf_hbm.at[pl.ds(base, R)])
        pltpu.sync_copy(accC.at[pl.ds(0, R)], sc_hbm.at[pl.ds(base, R)])

    return agg2(pk, srcl, dlocl, cnt)


def _rows(i, _=None):
    return (i, 0)


def _bcast(i, _=None):
    return (0, 0)


BLK = 512


def _tc_call(body, ins, blockable, out_widths):
    in_specs = [
        pl.BlockSpec((BLK, a.shape[1]), _rows) if b
        else pl.BlockSpec(a.shape, _bcast)
        for a, b in zip(ins, blockable)
    ]
    out_widths = [w if isinstance(w, tuple) else (w, _f32) for w in out_widths]
    out_shape = tuple(
        jax.ShapeDtypeStruct((NPAD, wd), dt) for wd, dt in out_widths)
    out_specs = tuple(pl.BlockSpec((BLK, wd), _rows) for wd, _ in out_widths)
    outs = pl.pallas_call(
        body,
        grid=(NPAD // BLK,),
        in_specs=in_specs,
        out_specs=out_specs,
        out_shape=out_shape,
    )(*ins)
    return outs


def _mm(a, b):
    return jnp.dot(a, b, preferred_element_type=_f32)


def _pk2(x):
    h = x.shape[1] // 2
    a = lax.bitcast_convert_type(
        x[:, :h].astype(jnp.bfloat16), jnp.uint16).astype(jnp.uint32)
    b = lax.bitcast_convert_type(
        x[:, h:].astype(jnp.bfloat16), jnp.uint16).astype(jnp.uint32)
    return lax.bitcast_convert_type(a | (b << 16), jnp.int32)


def kernel(feature, condition, edge_index,
           enc_f2h_W, enc_f2h_b, enc_c2h_W, enc_c2h_b, enc_h2h_W, enc_h2h_b,
           enc_mean_W, enc_mean_b, enc_logvar_W, enc_logvar_b,
           dec_z2h_W, dec_z2h_b, dec_c2h_W, dec_c2h_b, dec_h2h_W, dec_h2h_b,
           dec_out_W, dec_out_b):
    pad = NPAD - N
    fpad = jnp.pad(feature, ((0, pad), (0, 0)))
    cpad = jnp.pad(condition, ((0, pad), (0, 0)))
    noise = jax.random.normal(jax.random.key(1), (N, 64), _f32)
    npad_ = jnp.pad(noise, ((0, pad), (0, 0)))

    Whh1, Whh2 = enc_h2h_W[:128], enc_h2h_W[128:]
    Wdhh1, Wdhh2 = dec_h2h_W[:128], dec_h2h_W[128:]
    Wmlv = jnp.concatenate([enc_mean_W, enc_logvar_W], axis=1)
    bmlv = jnp.concatenate([enc_mean_b, enc_logvar_b]).reshape(1, 128)
    bf = enc_f2h_b.reshape(1, -1)
    bc = enc_c2h_b.reshape(1, -1)
    bh = enc_h2h_b.reshape(1, -1)
    bz = dec_z2h_b.reshape(1, -1)
    bdc = dec_c2h_b.reshape(1, -1)
    bdh = dec_h2h_b.reshape(1, -1)
    bout = dec_out_b.reshape(1, -1)

    deg, srcl, dlocl, cnt = _partition_call(edge_index[0], edge_index[1])
    degc = deg.reshape(NPAD, 1)

    def tc0(deg_r, f_r, c_r, dinv_o, pk_o):
        dv = lax.rsqrt(jnp.maximum(deg_r[...], 1.0))
        dinv_o[...] = dv
        fb = lax.bitcast_convert_type(
            (f_r[...] * dv).astype(jnp.bfloat16), jnp.uint16).astype(jnp.uint32)
        cb = lax.bitcast_convert_type(
            (c_r[...] * dv).astype(jnp.bfloat16), jnp.uint16).astype(jnp.uint32)
        pk_o[...] = lax.bitcast_convert_type(fb | (cb << 16), jnp.int32)

    dinv, pk = _tc_call(tc0, [degc, fpad, cpad], [True] * 3,
                        [1, (128, jnp.int32)])

    s_f, s_c = _dual_agg_call(pk, srcl, dlocl, cnt)

    def tc1(sf_r, sc_r, dv_r, wf, bf_r, wc, bc_r, wdc, bdc_r, whh1, whh2,
            wdhh2, ts_o, t2a_o):
        dv = dv_r[...]
        f2h = jnp.tanh(_mm(dv * sf_r[...], wf[...]) + bf_r[...])
        cpre = dv * sc_r[...]
        c2h = jnp.tanh(_mm(cpre, wc[...]) + bc_r[...])
        dc2h = jnp.tanh(_mm(cpre, wdc[...]) + bdc_r[...])
        ts_o[...] = _pk2(dv * (_mm(f2h, whh1[...]) + _mm(c2h, whh2[...])))
        t2a_o[...] = _mm(dc2h, wdhh2[...])

    ts, t2a = _tc_call(
        tc1,
        [s_f, s_c, dinv, enc_f2h_W, bf, enc_c2h_W, bc, dec_c2h_W, bdc,
         Whh1, Whh2, Wdhh2],
        [True, True, True] + [False] * 9,
        [(64, jnp.int32), 128])

    s_t = _make_agg_pk(128)(ts, srcl, dlocl, cnt)

    def tc2(st_r, dv_r, bh_r, wmlv, ms_o):
        dv = dv_r[...]
        h = jnp.tanh(dv * st_r[...] + bh_r[...])
        ms_o[...] = _pk2(dv * _mm(h, wmlv[...]))

    (ms,) = _tc_call(tc2, [s_t, dinv, bh, Wmlv],
                     [True, True, False, False], [(64, jnp.int32)])

    s_m = _make_agg_pk(128)(ms, srcl, dlocl, cnt)

    def tc3(sm_r, dv_r, bmlv_r, nz_r, mean_o, logvar_o, z_o, zs_o):
        dv = dv_r[...]
        mlv = dv * sm_r[...] + bmlv_r[...]
        mean = mlv[:, :64]
        logvar = mlv[:, 64:]
        z = nz_r[...] * jnp.exp(0.5 * logvar) + mean
        mean_o[...] = mean
        logvar_o[...] = logvar
        z_o[...] = z
        zs_o[...] = _pk2(dv * z)

    mean, logvar, z, zs = _tc_call(
        tc3, [s_m, dinv, bmlv, npad_],
        [True, True, False, True], [64, 64, 64, (32, jnp.int32)])

    s_z = _make_agg_pk(64)(zs, srcl, dlocl, cnt)

    def tc4(sz_r, dv_r, wz, bz_r, wdhh1, t2a_r, t2s_o):
        dv = dv_r[...]
        z2h = jnp.tanh(_mm(dv * sz_r[...], wz[...]) + bz_r[...])
        t2s_o[...] = _pk2(dv * (_mm(z2h, wdhh1[...]) + t2a_r[...]))

    (t2s,) = _tc_call(tc4, [s_z, dinv, dec_z2h_W, bz, Wdhh1, t2a],
                      [True, True, False, False, False, True],
                      [(64, jnp.int32)])

    s_t2 = _make_agg_pk(128)(t2s, srcl, dlocl, cnt)

    def tc5(st2_r, dv_r, bdh_r, wout, t3s_o):
        dv = dv_r[...]
        dh = jnp.tanh(dv * st2_r[...] + bdh_r[...])
        t3s_o[...] = _pk2(dv * _mm(dh, wout[...]))

    (t3s,) = _tc_call(tc5, [s_t2, dinv, bdh, dec_out_W],
                      [True, True, False, False], [(64, jnp.int32)])

    s_o = _make_agg_pk(128)(t3s, srcl, dlocl, cnt)

    def tc6(so_r, dv_r, bout_r, out_o):
        out_o[...] = dv_r[...] * so_r[...] + bout_r[...]

    (outp,) = _tc_call(tc6, [s_o, dinv, bout], [True, True, False], [128])

    return (z[:N], mean[:N], logvar[:N], outp[:N])


# speedup vs baseline: 1.4917x; 1.0002x over previous
import functools

import jax
import jax.numpy as jnp
from jax import lax
from jax.experimental import pallas as pl
from jax.experimental.pallas import tpu as pltpu
from jax.experimental.pallas import tpu_sc as plsc

N = 10000
E = 320000
NC, NS, L = 2, 16, 16
NW = NC * NS
R = 320
NPAD = NW * R
CAP = 16384
K = 128
ACCR = 336
DUMMY = 320
CE = 6400

_mesh = lambda: plsc.VectorSubcoreMesh(core_axis_name="c", subcore_axis_name="s")

_f32 = jnp.float32
_i32 = jnp.int32


def _wid():
    return lax.axis_index("s") * NC + lax.axis_index("c")


def _partition_call(src, dst):
    @functools.partial(
        pl.kernel,
        mesh=_mesh(),
        compiler_params=pltpu.CompilerParams(needs_layout_passes=False),
        out_type=(
            jax.ShapeDtypeStruct((NPAD,), _f32),
            jax.ShapeDtypeStruct((NW, CAP), _i32),
            jax.ShapeDtypeStruct((NW, CAP), _i32),
            jax.ShapeDtypeStruct((NW, L), _i32),
        ),
        scratch_types=[
            pltpu.VMEM((CE,), _i32),
            pltpu.VMEM((CE,), _i32),
            pltpu.VMEM((CE,), _i32),
            pltpu.VMEM((CE,), _i32),
            pltpu.VMEM((ACCR,), _f32),
            pltpu.VMEM((CAP,), _i32),
            pltpu.VMEM((CAP,), _i32),
            pltpu.VMEM((L,), _i32),
            pltpu.SemaphoreType.DMA,
            pltpu.SemaphoreType.DMA,
        ],
    )
    def p1(src_hbm, dst_hbm, deg_hbm, srcl_hbm, dlocl_hbm, cnt_hbm,
           sbuf0, dbuf0, sbuf1, dbuf1, dega, srca, dloca, cntv, sem0, sem1):
        iota = lax.iota(_i32, L)
        w = _wid()
        base = w * R
        nvalid = jnp.minimum(R, N - base)

        for i in range(ACCR // L):
            dega[pl.ds(i * L, L)] = jnp.zeros((L,), _f32)

        def issue(ci, sb, db, sem):
            pltpu.make_async_copy(src_hbm.at[pl.ds(ci * CE, CE)], sb, sem).start()
            pltpu.make_async_copy(dst_hbm.at[pl.ds(ci * CE, CE)], db, sem).start()

        def waitch(sb, db, sem):
            pltpu.make_async_copy(src_hbm.at[pl.ds(0, CE)], sb, sem).wait()
            pltpu.make_async_copy(dst_hbm.at[pl.ds(0, CE)], db, sem).wait()

        def scan_chunk(sb, db, offv):
            def grp(gi4, offv):
                g0 = gi4 * 4
                ss = [sb[pl.ds((g0 + k) * L, L)] for k in range(4)]
                dd = [db[pl.ds((g0 + k) * L, L)] for k in range(4)]
                dls_, ms_, cms_, pcs_ = [], [], [], []
                for k in range(4):
                    dl = dd[k] - base
                    m = (dl >= 0) & (dl < nvalid)
                    ms_.append(m)
                    dls_.append(jnp.where(m, dl, DUMMY))
                    cms_.append(plsc.cumsum(m.astype(_i32)))
                    pcs_.append(plsc.all_reduce_population_count(m))
                off = offv
                for k in range(4):
                    pos = jnp.where(ms_[k], off + cms_[k] - 1, CAP - L + iota)
                    plsc.addupdate_scatter(
                        dega, [dls_[k]], jnp.where(ms_[k], 1.0, 0.0))
                    plsc.store_scatter(srca, [pos], ss[k])
                    plsc.store_scatter(dloca, [pos], dls_[k])
                    off = off + pcs_[k]
                return jnp.minimum(off, CAP - 1024)

            return lax.fori_loop(0, CE // (4 * L), grp, offv)

        NCH = E // CE
        issue(0, sbuf0, dbuf0, sem0)

        def half(h, offv):
            i1 = 2 * h + 1
            issue(i1, sbuf1, dbuf1, sem1)
            waitch(sbuf0, dbuf0, sem0)
            offv = scan_chunk(sbuf0, dbuf0, offv)

            @pl.when(i1 + 1 < NCH)
            def _():
                issue(i1 + 1, sbuf0, dbuf0, sem0)

            waitch(sbuf1, dbuf1, sem1)
            offv = scan_chunk(sbuf1, dbuf1, offv)
            return offv

        offv = lax.fori_loop(0, NCH // 2, half, jnp.zeros((L,), _i32))
        off = jnp.max(offv)

        def slgrp(j, off):
            idxv = off + iota
            plsc.store_scatter(srca, [idxv], base + j * L + iota)
            plsc.store_scatter(dloca, [idxv], j * L + iota)
            cur = plsc.load_gather(dega, [j * L + iota])
            plsc.store_scatter(dega, [j * L + iota], cur + 1.0)
            return off + L

        off = lax.fori_loop(0, nvalid // L, slgrp, off)

        target = ((off + K - 1) // K) * K
        for i in range(K // L):
            idxv = off + i * L + iota
            idxv = jnp.where(idxv < target, idxv, CAP - L + iota)
            plsc.store_scatter(srca, [idxv], jnp.zeros((L,), _i32))
            plsc.store_scatter(dloca, [idxv], jnp.full((L,), DUMMY, _i32))

        cntv[...] = lax.broadcast(target // K, (L,))
        pltpu.sync_copy(cntv, cnt_hbm.at[w])
        pltpu.sync_copy(dega.at[pl.ds(0, R)], deg_hbm.at[pl.ds(base, R)])
        pltpu.sync_copy(srca, srcl_hbm.at[w])
        pltpu.sync_copy(dloca, dlocl_hbm.at[w])

    return p1(src, dst)


@functools.lru_cache(maxsize=None)
def _make_agg_pk(W):
    HW = W // 2

    @functools.partial(
        pl.kernel,
        mesh=_mesh(),
        compiler_params=pltpu.CompilerParams(
            needs_layout_passes=False, use_tc_tiling_on_sc=False),
        out_type=jax.ShapeDtypeStruct((NPAD, W), _f32),
        scratch_types=[
            pltpu.VMEM((CAP,), _i32),
            pltpu.VMEM((CAP,), _i32),
            pltpu.VMEM((L,), _i32),
            pltpu.VMEM((ACCR, W), _f32),
            pltpu.VMEM((K, HW), _i32),
            pltpu.VMEM((K, HW), _i32),
            pltpu.VMEM((K, HW), _i32),
            pltpu.SemaphoreType.DMA,
            pltpu.SemaphoreType.DMA,
            pltpu.SemaphoreType.DMA,
        ],
    )
    def agg(x_hbm, srcl_hbm, dlocl_hbm, cnt_hbm, s_hbm,
            srca, dloca, cntv, acc, rows0, rows1, rows2, sem0, sem1, sem2):
        iota = lax.iota(_i32, L)
        w = _wid()
        base = w * R
        pltpu.sync_copy(cnt_hbm.at[w], cntv)
        nc = jnp.max(cntv[...])
        pltpu.sync_copy(srcl_hbm.at[w], srca)
        pltpu.sync_copy(dlocl_hbm.at[w], dloca)

        def zrow(r, _):
            for j in range(W // L):
                acc[r, pl.ds(j * L, L)] = jnp.zeros((L,), _f32)
            return 0

        lax.fori_loop(0, ACCR, zrow, 0)

        bufs = ((rows0, sem0), (rows1, sem1), (rows2, sem2))

        def issue(i, b):
            pltpu.make_async_copy(
                x_hbm.at[srca.at[pl.ds(i * K, K)]], bufs[b][0], bufs[b][1]).start()

        def wait(b):
            pltpu.make_async_copy(
                x_hbm.at[srca.at[pl.ds(0, K)]], bufs[b][0], bufs[b][1]).wait()

        def process(i, rows):
            def grp(g, _):
                dl16 = dloca[pl.ds(i * K + g * L, L)]
                for j in range(0, L, 2):
                    r0 = dl16.at[lax.broadcast(j, (L,))].get(
                        mode="promise_in_bounds")
                    r1 = dl16.at[lax.broadcast(j + 1, (L,))].get(
                        mode="promise_in_bounds")
                    e0 = g * L + j
                    e1 = e0 + 1

                    def halves(e):
                        out = []
                        for c in range(HW // L):
                            v = rows[e, pl.ds(c * L, L)]
                            vb = plsc.bitcast(v, jnp.bfloat16)
                            out.append(plsc.unpack(
                                vb, format=plsc.PackFormat.INTERLEAVED))
                        return out

                    h0 = halves(e0)
                    h1 = halves(e1)
                    for c in range(HW // L):
                        plsc.addupdate_scatter(
                            acc, [r0, c * L + iota], h0[c][0])
                        plsc.addupdate_scatter(
                            acc, [r0, HW + c * L + iota], h0[c][1])
                    for c in range(HW // L):
                        plsc.addupdate_scatter(
                            acc, [r1, c * L + iota], h1[c][0])
                        plsc.addupdate_scatter(
                            acc, [r1, HW + c * L + iota], h1[c][1])
                return 0

            lax.fori_loop(0, K // L, grp, 0)

        issue(0, 0)

        @pl.when(1 < nc)
        def _():
            issue(1, 1)

        def trip(h, _):
            for b in range(3):
                i = 3 * h + b

                @pl.when(i < nc)
                def _():
                    wait(b)

                    @pl.when(i + 2 < nc)
                    def _():
                        issue(i + 2, (b + 2) % 3)

                    process(i, bufs[b][0])
            return 0

        lax.fori_loop(0, (nc + 2) // 3, trip, 0)
        pltpu.sync_copy(acc.at[pl.ds(0, R)], s_hbm.at[pl.ds(base, R)])

    return agg


K2 = 64


def _dual_agg_call(pk, srcl, dlocl, cnt):
    W = 128

    @functools.partial(
        pl.kernel,
        mesh=_mesh(),
        compiler_params=pltpu.CompilerParams(needs_layout_passes=False),
        out_type=(jax.ShapeDtypeStruct((NPAD, W), _f32),
                  jax.ShapeDtypeStruct((NPAD, W), _f32)),
        scratch_types=[
            pltpu.VMEM((CAP,), _i32),
            pltpu.VMEM((K2,), _i32),
            pltpu.VMEM((K2,), _i32),
            pltpu.VMEM((L,), _i32),
            pltpu.VMEM((ACCR, W), _f32),
            pltpu.VMEM((ACCR, W), _f32),
            pltpu.VMEM((K2, W), _i32),
            pltpu.VMEM((K2, W), _i32),
            pltpu.SemaphoreType.DMA,
            pltpu.SemaphoreType.DMA,
        ])
    def agg2(pk_hbm, srcl_hbm, dlocl_hbm, cnt_hbm, sf_hbm, sc_hbm,
             srca, dv0, dv1, cntv, accF, accC, rows0, rows1, sem0, sem1):
        iota = lax.iota(_i32, L)
        w = _wid()
        base = w * R
        pltpu.sync_copy(cnt_hbm.at[w], cntv)
        nc2 = jnp.max(cntv[...]) * 2
        pltpu.sync_copy(srcl_hbm.at[w], srca)

        def zrow(r, _):
            for j in range(W // L):
                accF[r, pl.ds(j * L, L)] = jnp.zeros((L,), _f32)
                accC[r, pl.ds(j * L, L)] = jnp.zeros((L,), _f32)
            return 0
        lax.fori_loop(0, ACCR, zrow, 0)

        def issue(i, rows, dv, sem):
            pltpu.make_async_copy(
                pk_hbm.at[srca.at[pl.ds(i * K2, K2)]], rows, sem).start()
            pltpu.make_async_copy(
                dlocl_hbm.at[w, pl.ds(i * K2, K2)], dv, sem).start()

        def wait(rows, dv, sem):
            pltpu.make_async_copy(
                pk_hbm.at[srca.at[pl.ds(0, K2)]], rows, sem).wait()
            pltpu.make_async_copy(
                dlocl_hbm.at[w, pl.ds(0, K2)], dv, sem).wait()

        def process(rows, dv):
            def grp(g, _):
                dl16 = dv[pl.ds(g * L, L)]
                for j in range(0, L, 2):
                    r0 = dl16.at[lax.broadcast(j, (L,))].get(
                        mode="promise_in_bounds")
                    r1 = dl16.at[lax.broadcast(j + 1, (L,))].get(
                        mode="promise_in_bounds")
                    e0 = g * L + j
                    e1 = e0 + 1

                    def halves(e):
                        out = []
                        for c in range(W // L):
                            v = rows[e, pl.ds(c * L, L)]
                            vb = plsc.bitcast(v, jnp.bfloat16)
                            out.append(plsc.unpack(
                                vb, format=plsc.PackFormat.INTERLEAVED))
                        return out

                    h0 = halves(e0)
                    h1 = halves(e1)
                    for c in range(W // L):
                        plsc.addupdate_scatter(
                            accF, [r0, c * L + iota], h0[c][0])
                        plsc.addupdate_scatter(
                            accC, [r0, c * L + iota], h0[c][1])
                    for c in range(W // L):
                        plsc.addupdate_scatter(
                            accF, [r1, c * L + iota], h1[c][0])
                        plsc.addupdate_scatter(
                            accC, [r1, c * L + iota], h1[c][1])
                return 0
            lax.fori_loop(0, K2 // L, grp, 0)

        issue(0, rows0, dv0, sem0)

        def half(h, _):
            i0, i1 = 2 * h, 2 * h + 1

            @pl.when(i1 < nc2)
            def _():
                issue(i1, rows1, dv1, sem1)
            wait(rows0, dv0, sem0)
            process(rows0, dv0)

            @pl.when(i1 < nc2)
            def _():
                @pl.when(i1 + 1 < nc2)
                def _():
                    issue(i1 + 1, rows0, dv0, sem0)
                wait(rows1, dv1, sem1)
                process(rows1, dv1)
            return 0

        lax.fori_loop(0, (nc2 + 1) // 2, half, 0)
        pltpu.sync_copy(accF.at[pl.ds(0, R)], sf_hbm.at[pl.ds(base, R)])
        pltpu.sync_copy(accC.at[pl.ds(0, R)], sc_hbm.at[pl.ds(base, R)])

    return agg2(pk, srcl, dlocl, cnt)


def _rows(i, _=None):
    return (i, 0)


def _bcast(i, _=None):
    return (0, 0)


BLK = 512


def _tc_call(body, ins, blockable, out_widths):
    in_specs = [
        pl.BlockSpec((BLK, a.shape[1]), _rows) if b
        else pl.BlockSpec(a.shape, _bcast)
        for a, b in zip(ins, blockable)
    ]
    out_widths = [w if isinstance(w, tuple) else (w, _f32) for w in out_widths]
    out_shape = tuple(
        jax.ShapeDtypeStruct((NPAD, wd), dt) for wd, dt in out_widths)
    out_specs = tuple(pl.BlockSpec((BLK, wd), _rows) for wd, _ in out_widths)
    outs = pl.pallas_call(
        body,
        grid=(NPAD // BLK,),
        in_specs=in_specs,
        out_specs=out_specs,
        out_shape=out_shape,
    )(*ins)
    return outs


def _mm(a, b):
    return jnp.dot(a, b, preferred_element_type=_f32)


def _pk2(x):
    h = x.shape[1] // 2
    a = lax.bitcast_convert_type(
        x[:, :h].astype(jnp.bfloat16), jnp.uint16).astype(jnp.uint32)
    b = lax.bitcast_convert_type(
        x[:, h:].astype(jnp.bfloat16), jnp.uint16).astype(jnp.uint32)
    return lax.bitcast_convert_type(a | (b << 16), jnp.int32)


def kernel(feature, condition, edge_index,
           enc_f2h_W, enc_f2h_b, enc_c2h_W, enc_c2h_b, enc_h2h_W, enc_h2h_b,
           enc_mean_W, enc_mean_b, enc_logvar_W, enc_logvar_b,
           dec_z2h_W, dec_z2h_b, dec_c2h_W, dec_c2h_b, dec_h2h_W, dec_h2h_b,
           dec_out_W, dec_out_b):
    pad = NPAD - N
    fpad = jnp.pad(feature, ((0, pad), (0, 0)))
    cpad = jnp.pad(condition, ((0, pad), (0, 0)))
    noise = jax.random.normal(jax.random.key(1), (N, 64), _f32)
    npad_ = jnp.pad(noise, ((0, pad), (0, 0)))

    Whh1, Whh2 = enc_h2h_W[:128], enc_h2h_W[128:]
    Wdhh1, Wdhh2 = dec_h2h_W[:128], dec_h2h_W[128:]
    Wmlv = jnp.concatenate([enc_mean_W, enc_logvar_W], axis=1)
    bmlv = jnp.concatenate([enc_mean_b, enc_logvar_b]).reshape(1, 128)
    bf = enc_f2h_b.reshape(1, -1)
    bc = enc_c2h_b.reshape(1, -1)
    bh = enc_h2h_b.reshape(1, -1)
    bz = dec_z2h_b.reshape(1, -1)
    bdc = dec_c2h_b.reshape(1, -1)
    bdh = dec_h2h_b.reshape(1, -1)
    bout = dec_out_b.reshape(1, -1)

    deg, srcl, dlocl, cnt = _partition_call(edge_index[0], edge_index[1])
    degc = deg.reshape(NPAD, 1)

    def tc0(deg_r, f_r, c_r, dinv_o, pk_o):
        dv = lax.rsqrt(jnp.maximum(deg_r[...], 1.0))
        dinv_o[...] = dv
        fb = lax.bitcast_convert_type(
            (f_r[...] * dv).astype(jnp.bfloat16), jnp.uint16).astype(jnp.uint32)
        cb = lax.bitcast_convert_type(
            (c_r[...] * dv).astype(jnp.bfloat16), jnp.uint16).astype(jnp.uint32)
        pk_o[...] = lax.bitcast_convert_type(fb | (cb << 16), jnp.int32)

    dinv, pk = _tc_call(tc0, [degc, fpad, cpad], [True] * 3,
                        [1, (128, jnp.int32)])

    s_f, s_c = _dual_agg_call(pk, srcl, dlocl, cnt)

    def tc1(sf_r, sc_r, dv_r, wf, bf_r, wc, bc_r, wdc, bdc_r, whh1, whh2,
            wdhh2, ts_o, t2a_o):
        dv = dv_r[...]
        f2h = jnp.tanh(_mm(dv * sf_r[...], wf[...]) + bf_r[...])
        cpre = dv * sc_r[...]
        c2h = jnp.tanh(_mm(cpre, wc[...]) + bc_r[...])
        dc2h = jnp.tanh(_mm(cpre, wdc[...]) + bdc_r[...])
        ts_o[...] = _pk2(dv * (_mm(f2h, whh1[...]) + _mm(c2h, whh2[...])))
        t2a_o[...] = _mm(dc2h, wdhh2[...])

    ts, t2a = _tc_call(
        tc1,
        [s_f, s_c, dinv, enc_f2h_W, bf, enc_c2h_W, bc, dec_c2h_W, bdc,
         Whh1, Whh2, Wdhh2],
        [True, True, True] + [False] * 9,
        [(64, jnp.int32), 128])

    s_t = _make_agg_pk(128)(ts, srcl, dlocl, cnt)

    def tc2(st_r, dv_r, bh_r, wmlv, ms_o):
        dv = dv_r[...]
        h = jnp.tanh(dv * st_r[...] + bh_r[...])
        ms_o[...] = _pk2(dv * _mm(h, wmlv[...]))

    (ms,) = _tc_call(tc2, [s_t, dinv, bh, Wmlv],
                     [True, True, False, False], [(64, jnp.int32)])

    s_m = _make_agg_pk(128)(ms, srcl, dlocl, cnt)

    def tc3(sm_r, dv_r, bmlv_r, nz_r, mean_o, logvar_o, z_o, zs_o):
        dv = dv_r[...]
        mlv = dv * sm_r[...] + bmlv_r[...]
        mean = mlv[:, :64]
        logvar = mlv[:, 64:]
        z = nz_r[...] * jnp.exp(0.5 * logvar) + mean
        mean_o[...] = mean
        logvar_o[...] = logvar
        z_o[...] = z
        zs_o[...] = _pk2(dv * z)

    mean, logvar, z, zs = _tc_call(
        tc3, [s_m, dinv, bmlv, npad_],
        [True, True, False, True], [64, 64, 64, (32, jnp.int32)])

    s_z = _make_agg_pk(64)(zs, srcl, dlocl, cnt)

    def tc4(sz_r, dv_r, wz, bz_r, wdhh1, t2a_r, t2s_o):
        dv = dv_r[...]
        z2h = jnp.tanh(_mm(dv * sz_r[...], wz[...]) + bz_r[...])
        t2s_o[...] = _pk2(dv * (_mm(z2h, wdhh1[...]) + t2a_r[...]))

    (t2s,) = _tc_call(tc4, [s_z, dinv, dec_z2h_W, bz, Wdhh1, t2a],
                      [True, True, False, False, False, True],
                      [(64, jnp.int32)])

    s_t2 = _make_agg_pk(128)(t2s, srcl, dlocl, cnt)

    def tc5(st2_r, dv_r, bdh_r, wout, t3s_o):
        dv = dv_r[...]
        dh = jnp.tanh(dv * st2_r[...] + bdh_r[...])
        t3s_o[...] = _pk2(dv * _mm(dh, wout[...]))

    (t3s,) = _tc_call(tc5, [s_t2, dinv, bdh, dec_out_W],
                      [True, True, False, False], [(64, jnp.int32)])

    s_o = _make_agg_pk(128)(t3s, srcl, dlocl, cnt)

    def tc6(so_r, dv_r, bout_r, out_o):
        out_o[...] = dv_r[...] * so_r[...] + bout_r[...]

    (outp,) = _tc_call(tc6, [s_o, dinv, bout], [True, True, False], [128])

    return (z[:N], mean[:N], logvar[:N], outp[:N])
